# baseline (speedup 1.0000x reference)
import functools

import jax
import jax.numpy as jnp
from jax import lax
from jax.experimental import pallas as pl
from jax.experimental.pallas import tpu as pltpu
from jax.experimental.pallas import tpu_sc as plsc

N = 10000
E = 320000
DH = 64
NC = 2
NS = 16
NW = NC * NS
KCH = 64
NCHUNK = 160
PH = 2
CPP = NCHUNK // PH
EPAD = NW * NCHUNK * KCH
TOTCH = EPAD // KCH
NP = N + 16
RPT = 624
TAIL0 = RPT * NS
TAILN = NP - TAIL0

ROW_BLK = 1000
GRID = N // ROW_BLK


def _make_prop(d):
    mesh = plsc.VectorSubcoreMesh(
        core_axis_name="c", subcore_axis_name="s", num_cores=NC, num_subcores=NS
    )

    @functools.partial(
        pl.kernel,
        out_type=jax.ShapeDtypeStruct((NC * NP, d), jnp.float32),
        mesh=mesh,
        scratch_types=[
            pltpu.VMEM_SHARED((NP, d), jnp.float32),
            pltpu.VMEM((KCH,), jnp.int32),
            pltpu.VMEM((KCH,), jnp.int32),
            pltpu.VMEM((KCH, d), jnp.float32),
            pltpu.VMEM((KCH, d), jnp.float32),
            pltpu.SemaphoreType.DMA,
            pltpu.SemaphoreType.DMA,
        ],
    )
    def prop(table, idxg, idxs, zeros, out, acc, gidx_v, sidx_v, rows_a, rows_b,
             sem_a, sem_b):
        cid = lax.axis_index("c")
        sid = lax.axis_index("s")
        wid = cid * NS + sid
        r0 = sid * RPT
        c0 = wid * NCHUNK
        pltpu.sync_copy(zeros, acc.at[pl.ds(r0, RPT)])

        @pl.when(sid == NS - 1)
        def _():
            pltpu.sync_copy(zeros.at[pl.ds(0, TAILN)], acc.at[pl.ds(TAIL0, TAILN)])

        plsc.subcore_barrier()

        e0 = wid * NCHUNK * KCH

        def body(j, carry):
            off = pl.multiple_of(e0 + j * KCH, 8)
            pltpu.sync_copy(idxg.at[pl.ds(off, KCH)], gidx_v)
            pltpu.sync_copy(idxs.at[pl.ds(off, KCH)], sidx_v)
            pltpu.async_copy(table.at[gidx_v], rows_a, sem_a).wait()
            pltpu.sync_copy(rows_a, acc.at[sidx_v], add=True)
            return carry

        lax.fori_loop(0, NCHUNK, body, 0)
        plsc.subcore_barrier()
        pltpu.sync_copy(
            acc.at[pl.ds(r0, RPT)], out.at[pl.ds(cid * NP + r0, RPT)]
        )

        @pl.when(sid == NS - 1)
        def _():
            pltpu.sync_copy(
                acc.at[pl.ds(TAIL0, TAILN)], out.at[pl.ds(cid * NP + TAIL0, TAILN)]
            )

    return prop


_prop128 = _make_prop(128)


def _row_spec(cols):
    return pl.BlockSpec((ROW_BLK, cols), lambda i: (i, 0))


def _pair_spec(cols):
    return pl.BlockSpec((ROW_BLK, cols), lambda i: (i, 0))


def _full_spec(rows, cols):
    return pl.BlockSpec((rows, cols), lambda i: (0, 0))


def _tc_call(body, in_specs, out_shape, out_specs):
    return pl.pallas_call(
        body,
        grid=(GRID,),
        in_specs=in_specs,
        out_shape=out_shape,
        out_specs=out_specs,
    )


def _enc_body(x_ref, w_ref, b_ref, y_ref):
    y = jnp.dot(x_ref[...], w_ref[...], preferred_element_type=jnp.float32)
    y_ref[...] = jnp.maximum(y + b_ref[...], 0.0)


def _stage1_body(xr, yr, w1r, dvr, ur):
    acc = jnp.dot(xr[...], w1r[:DH], preferred_element_type=jnp.float32)
    acc += jnp.dot(yr[...], w1r[DH:], preferred_element_type=jnp.float32)
    ur[...] = dvr[...] * acc


def _stage2_body(pa, pb, ur, dvr, b1r, w2pr, o1r, u1r):
    o1 = jnp.tanh(dvr[...] * (pa[...] + pb[...] + ur[...]) + b1r[...])
    o1r[...] = o1
    u1r[...] = dvr[...] * jnp.dot(o1, w2pr[...], preferred_element_type=jnp.float32)


def _stage3_body(pa, pb, u1r, dvr, b2r, cr, w3pr, v2r):
    o2 = jnp.tanh(dvr[...] * (pa[...] + pb[...] + u1r[...]) + b2r[...])
    v2r[...] = dvr[...] * (1.0 - o2 * o2) * (cr[...] * w3pr[...])


def _stage4_body(qa, qb, v2r, dvr, o1r, w2pr, v1r):
    t = dvr[...] * (qa[...] + qb[...] + v2r[...])
    go1 = jnp.dot(t, w2pr[...].T, preferred_element_type=jnp.float32)
    o1 = o1r[...]
    v1r[...] = dvr[...] * (1.0 - o1 * o1) * go1


def _stage5_body(qa, qb, v1r, dvr, w1r, xr, yr, xnr, ynr):
    z = dvr[...] * (qa[...] + qb[...] + v1r[...])
    xnr[...] = xr[...] + jnp.dot(z, w1r[DH:].T, preferred_element_type=jnp.float32)
    ynr[...] = yr[...] - jnp.dot(z, w1r[:DH].T, preferred_element_type=jnp.float32)


def _dec_body(xr, wr, br, outr):
    outr[...] = jnp.dot(xr[...], wr[...], preferred_element_type=jnp.float32) + br[...]


def kernel(x, edge_index, W_enc, b_enc, W1, b1, W2, b2, W3, b3, W_dec, b_dec):
    f32 = jnp.float32
    src = edge_index[0]
    dst = edge_index[1]
    z128 = jnp.zeros((RPT, 128), f32)
    ones128 = jnp.ones((N, 128), f32)

    npad = EPAD - E
    gpad = jnp.zeros((npad,), jnp.int32)
    spad = jnp.full((npad,), N, jnp.int32)
    g_fwd = jnp.concatenate([src, gpad])
    s_fwd = jnp.concatenate([dst, spad])
    g_bwd = jnp.concatenate([dst, gpad])
    s_bwd = jnp.concatenate([src, spad])

    degp = _prop128(ones128, g_fwd, s_fwd, z128)
    deg = degp[:N, 0] + degp[NP:NP + N, 0] + 1.0
    dinv = lax.rsqrt(deg)
    dinv128 = jnp.broadcast_to(dinv[:, None], (N, 128))
    ctp = _prop128(dinv128, g_bwd, s_bwd, z128)
    c0 = dinv * (ctp[:N, 0] + ctp[NP:NP + N, 0] + dinv)

    dv2 = dinv[:, None]
    c2 = c0[:, None]
    b1_ = b1[None, :]
    b2p = jnp.concatenate([b2, jnp.zeros((DH,), f32)])[None, :]
    benc_ = b_enc[None, :]
    bdec_ = b_dec[None, :]
    W2p = jnp.concatenate([W2, jnp.zeros((128, DH), f32)], axis=1)
    w3p = jnp.concatenate([W3[:, 0], jnp.zeros((DH,), f32)])[None, :]

    sc_dv = pl.BlockSpec((ROW_BLK, 1), lambda i: (i, 0))

    Y = _tc_call(
        _enc_body,
        [_row_spec(128), _full_spec(128, DH), _full_spec(1, DH)],
        jax.ShapeDtypeStruct((N, DH), f32),
        _row_spec(DH),
    )(x, W_enc, benc_)
    X = Y

    stage1 = _tc_call(
        _stage1_body,
        [_row_spec(DH), _row_spec(DH), _full_spec(128, 128), sc_dv],
        jax.ShapeDtypeStruct((N, 128), f32),
        _row_spec(128),
    )
    stage2 = _tc_call(
        _stage2_body,
        [_row_spec(128), _row_spec(128), _row_spec(128), sc_dv,
         _full_spec(1, 128), _full_spec(128, 128)],
        [jax.ShapeDtypeStruct((N, 128), f32), jax.ShapeDtypeStruct((N, 128), f32)],
        [_row_spec(128), _row_spec(128)],
    )
    stage3 = _tc_call(
        _stage3_body,
        [_row_spec(128), _row_spec(128), _row_spec(128), sc_dv,
         _full_spec(1, 128), sc_dv, _full_spec(1, 128)],
        jax.ShapeDtypeStruct((N, 128), f32),
        _row_spec(128),
    )
    stage4 = _tc_call(
        _stage4_body,
        [_row_spec(128), _row_spec(128), _row_spec(128), sc_dv,
         _row_spec(128), _full_spec(128, 128)],
        jax.ShapeDtypeStruct((N, 128), f32),
        _row_spec(128),
    )
    stage5 = _tc_call(
        _stage5_body,
        [_row_spec(128), _row_spec(128), _row_spec(128), sc_dv,
         _full_spec(128, 128), _row_spec(DH), _row_spec(DH)],
        [jax.ShapeDtypeStruct((N, DH), f32), jax.ShapeDtypeStruct((N, DH), f32)],
        [_row_spec(DH), _row_spec(DH)],
    )

    for _ in range(2):
        u0 = stage1(X, Y, W1, dv2)
        p0 = _prop128(u0, g_fwd, s_fwd, z128)
        o1, u1 = stage2(p0[:N], p0[NP:NP + N], u0, dv2, b1_, W2p)
        p1 = _prop128(u1, g_fwd, s_fwd, z128)
        v2 = stage3(p1[:N], p1[NP:NP + N], u1, dv2, b2p, c2, w3p)
        q2 = _prop128(v2, g_bwd, s_bwd, z128)
        v1 = stage4(q2[:N], q2[NP:NP + N], v2, dv2, o1, W2p)
        q1 = _prop128(v1, g_bwd, s_bwd, z128)
        X, Y = stage5(q1[:N], q1[NP:NP + N], v1, dv2, W1, X, Y)

    out = _tc_call(
        _dec_body,
        [_row_spec(DH), _full_spec(DH, 16), _full_spec(1, 16)],
        jax.ShapeDtypeStruct((N, 16), f32),
        _row_spec(16),
    )(X, W_dec, bdec_)
    return out


# speedup vs baseline: 1.0607x; 1.0607x over previous
import functools

import jax
import jax.numpy as jnp
from jax import lax
from jax.experimental import pallas as pl
from jax.experimental.pallas import tpu as pltpu
from jax.experimental.pallas import tpu_sc as plsc

N = 10000
E = 320000
DH = 64
NC = 2
NS = 16
NW = NC * NS
KCH = 80
NCHUNK = 128
EPAD = NW * NCHUNK * KCH
NP = N + 1008
RPT = NP // NS

ROW_BLK = 1000
GRID = N // ROW_BLK


def _make_prop(d):
    mesh = plsc.VectorSubcoreMesh(
        core_axis_name="c", subcore_axis_name="s", num_cores=NC, num_subcores=NS
    )

    @functools.partial(
        pl.kernel,
        out_type=jax.ShapeDtypeStruct((NC * NP, d), jnp.float32),
        mesh=mesh,
        scratch_types=[
            pltpu.VMEM_SHARED((NP, d), jnp.float32),
            pltpu.VMEM((KCH,), jnp.int32),
            pltpu.VMEM((KCH,), jnp.int32),
            pltpu.VMEM((KCH, d), jnp.float32),
            pltpu.VMEM((KCH, d), jnp.float32),
            pltpu.SemaphoreType.DMA,
            pltpu.SemaphoreType.DMA,
        ],
    )
    def prop(table, idxg, idxs, zeros, out, acc, gidx_v, sidx_v, rows_a, rows_b,
             sem_a, sem_b):
        cid = lax.axis_index("c")
        sid = lax.axis_index("s")
        wid = cid * NS + sid
        r0 = sid * RPT
        pltpu.sync_copy(zeros, acc.at[pl.ds(r0, RPT)])
        plsc.subcore_barrier()

        e0 = wid * NCHUNK * KCH

        def body(j, carry):
            off = pl.multiple_of(e0 + j * KCH, 8)
            pltpu.sync_copy(idxg.at[pl.ds(off, KCH)], gidx_v)
            pltpu.sync_copy(idxs.at[pl.ds(off, KCH)], sidx_v)
            pltpu.async_copy(table.at[gidx_v], rows_a, sem_a).wait()
            pltpu.sync_copy(rows_a, acc.at[sidx_v], add=True)
            return carry

        lax.fori_loop(0, NCHUNK, body, 0)
        plsc.subcore_barrier()
        pltpu.sync_copy(
            acc.at[pl.ds(r0, RPT)], out.at[pl.ds(cid * NP + r0, RPT)]
        )

    return prop


_prop128 = _make_prop(128)


def _row_spec(cols):
    return pl.BlockSpec((ROW_BLK, cols), lambda i: (i, 0))


def _pair_spec(cols):
    return pl.BlockSpec((ROW_BLK, cols), lambda i: (i, 0))


def _full_spec(rows, cols):
    return pl.BlockSpec((rows, cols), lambda i: (0, 0))


def _tc_call(body, in_specs, out_shape, out_specs):
    return pl.pallas_call(
        body,
        grid=(GRID,),
        in_specs=in_specs,
        out_shape=out_shape,
        out_specs=out_specs,
    )


def _enc_body(x_ref, w_ref, b_ref, y_ref):
    y = jnp.dot(x_ref[...], w_ref[...], preferred_element_type=jnp.float32)
    y_ref[...] = jnp.maximum(y + b_ref[...], 0.0)


def _stage1_body(xr, yr, w1r, dvr, ur):
    acc = jnp.dot(xr[...], w1r[:DH], preferred_element_type=jnp.float32)
    acc += jnp.dot(yr[...], w1r[DH:], preferred_element_type=jnp.float32)
    ur[...] = dvr[...] * acc


def _stage2_body(pa, pb, ur, dvr, b1r, w2pr, o1r, u1r):
    o1 = jnp.tanh(dvr[...] * (pa[...] + pb[...] + ur[...]) + b1r[...])
    o1r[...] = o1
    u1r[...] = dvr[...] * jnp.dot(o1, w2pr[...], preferred_element_type=jnp.float32)


def _stage3_body(pa, pb, u1r, dvr, b2r, cr, w3pr, v2r):
    o2 = jnp.tanh(dvr[...] * (pa[...] + pb[...] + u1r[...]) + b2r[...])
    v2r[...] = dvr[...] * (1.0 - o2 * o2) * (cr[...] * w3pr[...])


def _stage4_body(qa, qb, v2r, dvr, o1r, w2pr, v1r):
    t = dvr[...] * (qa[...] + qb[...] + v2r[...])
    go1 = jnp.dot(t, w2pr[...].T, preferred_element_type=jnp.float32)
    o1 = o1r[...]
    v1r[...] = dvr[...] * (1.0 - o1 * o1) * go1


def _stage5_body(qa, qb, v1r, dvr, w1r, xr, yr, xnr, ynr):
    z = dvr[...] * (qa[...] + qb[...] + v1r[...])
    xnr[...] = xr[...] + jnp.dot(z, w1r[DH:].T, preferred_element_type=jnp.float32)
    ynr[...] = yr[...] - jnp.dot(z, w1r[:DH].T, preferred_element_type=jnp.float32)


def _dec_body(xr, wr, br, outr):
    outr[...] = jnp.dot(xr[...], wr[...], preferred_element_type=jnp.float32) + br[...]


def kernel(x, edge_index, W_enc, b_enc, W1, b1, W2, b2, W3, b3, W_dec, b_dec):
    f32 = jnp.float32
    src = edge_index[0]
    dst = edge_index[1]
    z128 = jnp.zeros((RPT, 128), f32)
    ones128 = jnp.ones((N, 128), f32)

    npad = EPAD - E
    gpad = jnp.zeros((npad,), jnp.int32)
    spad = N + (jnp.arange(npad, dtype=jnp.int32) % (NP - N))
    g_fwd = jnp.concatenate([src, gpad])
    s_fwd = jnp.concatenate([dst, spad])
    g_bwd = jnp.concatenate([dst, gpad])
    s_bwd = jnp.concatenate([src, spad])

    degp = _prop128(ones128, g_fwd, s_fwd, z128)
    deg = degp[:N, 0] + degp[NP:NP + N, 0] + 1.0
    dinv = lax.rsqrt(deg)
    dinv128 = jnp.broadcast_to(dinv[:, None], (N, 128))
    ctp = _prop128(dinv128, g_bwd, s_bwd, z128)
    c0 = dinv * (ctp[:N, 0] + ctp[NP:NP + N, 0] + dinv)

    dv2 = dinv[:, None]
    c2 = c0[:, None]
    b1_ = b1[None, :]
    b2p = jnp.concatenate([b2, jnp.zeros((DH,), f32)])[None, :]
    benc_ = b_enc[None, :]
    bdec_ = b_dec[None, :]
    W2p = jnp.concatenate([W2, jnp.zeros((128, DH), f32)], axis=1)
    w3p = jnp.concatenate([W3[:, 0], jnp.zeros((DH,), f32)])[None, :]

    sc_dv = pl.BlockSpec((ROW_BLK, 1), lambda i: (i, 0))

    Y = _tc_call(
        _enc_body,
        [_row_spec(128), _full_spec(128, DH), _full_spec(1, DH)],
        jax.ShapeDtypeStruct((N, DH), f32),
        _row_spec(DH),
    )(x, W_enc, benc_)
    X = Y

    stage1 = _tc_call(
        _stage1_body,
        [_row_spec(DH), _row_spec(DH), _full_spec(128, 128), sc_dv],
        jax.ShapeDtypeStruct((N, 128), f32),
        _row_spec(128),
    )
    stage2 = _tc_call(
        _stage2_body,
        [_row_spec(128), _row_spec(128), _row_spec(128), sc_dv,
         _full_spec(1, 128), _full_spec(128, 128)],
        [jax.ShapeDtypeStruct((N, 128), f32), jax.ShapeDtypeStruct((N, 128), f32)],
        [_row_spec(128), _row_spec(128)],
    )
    stage3 = _tc_call(
        _stage3_body,
        [_row_spec(128), _row_spec(128), _row_spec(128), sc_dv,
         _full_spec(1, 128), sc_dv, _full_spec(1, 128)],
        jax.ShapeDtypeStruct((N, 128), f32),
        _row_spec(128),
    )
    stage4 = _tc_call(
        _stage4_body,
        [_row_spec(128), _row_spec(128), _row_spec(128), sc_dv,
         _row_spec(128), _full_spec(128, 128)],
        jax.ShapeDtypeStruct((N, 128), f32),
        _row_spec(128),
    )
    stage5 = _tc_call(
        _stage5_body,
        [_row_spec(128), _row_spec(128), _row_spec(128), sc_dv,
         _full_spec(128, 128), _row_spec(DH), _row_spec(DH)],
        [jax.ShapeDtypeStruct((N, DH), f32), jax.ShapeDtypeStruct((N, DH), f32)],
        [_row_spec(DH), _row_spec(DH)],
    )

    for _ in range(2):
        u0 = stage1(X, Y, W1, dv2)
        p0 = _prop128(u0, g_fwd, s_fwd, z128)
        o1, u1 = stage2(p0[:N], p0[NP:NP + N], u0, dv2, b1_, W2p)
        p1 = _prop128(u1, g_fwd, s_fwd, z128)
        v2 = stage3(p1[:N], p1[NP:NP + N], u1, dv2, b2p, c2, w3p)
        q2 = _prop128(v2, g_bwd, s_bwd, z128)
        v1 = stage4(q2[:N], q2[NP:NP + N], v2, dv2, o1, W2p)
        q1 = _prop128(v1, g_bwd, s_bwd, z128)
        X, Y = stage5(q1[:N], q1[NP:NP + N], v1, dv2, W1, X, Y)

    out = _tc_call(
        _dec_body,
        [_row_spec(DH), _full_spec(DH, 16), _full_spec(1, 16)],
        jax.ShapeDtypeStruct((N, 16), f32),
        _row_spec(16),
    )(X, W_dec, bdec_)
    return out


# speedup vs baseline: 2.1223x; 2.0008x over previous
import functools

import jax
import jax.numpy as jnp
from jax import lax
from jax.experimental import pallas as pl
from jax.experimental.pallas import tpu as pltpu
from jax.experimental.pallas import tpu_sc as plsc

N = 10000
E = 320000
DH = 64
NC = 2
NS = 16
NW = NC * NS
KCH = 80
NCHUNK = 128
EPAD = NW * NCHUNK * KCH
NP = N + 1008
RPT = NP // NS

ROW_BLK = 1000
GRID = N // ROW_BLK


def _make_prop(d):
    mesh = plsc.VectorSubcoreMesh(
        core_axis_name="c", subcore_axis_name="s", num_cores=NC, num_subcores=NS
    )

    @functools.partial(
        pl.kernel,
        out_type=jax.ShapeDtypeStruct((NC * NP, d), jnp.float32),
        mesh=mesh,
        scratch_types=[
            pltpu.VMEM_SHARED((NP, d), jnp.float32),
            pltpu.VMEM((KCH,), jnp.int32),
            pltpu.VMEM((KCH,), jnp.int32),
            pltpu.VMEM((KCH, d), jnp.float32),
            pltpu.VMEM((KCH, d), jnp.float32),
            pltpu.SemaphoreType.DMA,
            pltpu.SemaphoreType.DMA,
        ],
    )
    def prop(table, idxg, idxs, zeros, out, acc, gidx_v, sidx_v, rows_a, rows_b,
             sem_a, sem_b):
        cid = lax.axis_index("c")
        sid = lax.axis_index("s")
        wid = cid * NS + sid
        r0 = sid * RPT
        pltpu.sync_copy(zeros, acc.at[pl.ds(r0, RPT)])
        plsc.subcore_barrier()

        e0 = wid * NCHUNK * KCH

        def body(j, carry):
            off = pl.multiple_of(e0 + j * KCH, 8)
            pltpu.sync_copy(idxg.at[pl.ds(off, KCH)], gidx_v)
            pltpu.sync_copy(idxs.at[pl.ds(off, KCH)], sidx_v)
            pltpu.async_copy(table.at[gidx_v], rows_a, sem_a).wait()
            pltpu.sync_copy(rows_a, acc.at[sidx_v], add=True)
            return carry

        lax.fori_loop(0, NCHUNK, body, 0)
        plsc.subcore_barrier()
        pltpu.sync_copy(
            acc.at[pl.ds(r0, RPT)], out.at[pl.ds(cid * NP + r0, RPT)]
        )

    return prop


_prop128 = _make_prop(128)


def _row_spec(cols):
    return pl.BlockSpec((ROW_BLK, cols), lambda i: (i, 0))


def _pair_spec(cols):
    return pl.BlockSpec((ROW_BLK, cols), lambda i: (i, 0))


def _full_spec(rows, cols):
    return pl.BlockSpec((rows, cols), lambda i: (0, 0))


def _tc_call(body, in_specs, out_shape, out_specs):
    return pl.pallas_call(
        body,
        grid=(GRID,),
        in_specs=in_specs,
        out_shape=out_shape,
        out_specs=out_specs,
    )


def _enc_body(x_ref, w_ref, b_ref, y_ref):
    y = jnp.dot(x_ref[...], w_ref[...], preferred_element_type=jnp.float32)
    y_ref[...] = jnp.maximum(y + b_ref[...], 0.0)


def _stage1_body(xr, yr, w1r, dvr, ur):
    acc = jnp.dot(xr[...], w1r[:DH], preferred_element_type=jnp.float32)
    acc += jnp.dot(yr[...], w1r[DH:], preferred_element_type=jnp.float32)
    ur[...] = dvr[...] * acc


def _stage2_body(pa, pb, ur, dvr, b1r, w2pr, o1r, u1r):
    o1 = jnp.tanh(dvr[...] * (pa[...] + pb[...] + ur[...]) + b1r[...])
    o1r[...] = o1
    u1r[...] = dvr[...] * jnp.dot(o1, w2pr[...], preferred_element_type=jnp.float32)


def _stage3_body(pa, pb, u1r, dvr, b2r, cr, w3pr, v2r):
    o2 = jnp.tanh(dvr[...] * (pa[...] + pb[...] + u1r[...]) + b2r[...])
    v2r[...] = dvr[...] * (1.0 - o2 * o2) * (cr[...] * w3pr[...])


def _stage4_body(qa, qb, v2r, dvr, o1r, w2pr, v1r):
    t = dvr[...] * (qa[...] + qb[...] + v2r[...])
    go1 = jnp.dot(t, w2pr[...].T, preferred_element_type=jnp.float32)
    o1 = o1r[...]
    v1r[...] = dvr[...] * (1.0 - o1 * o1) * go1


def _stage5_body(qa, qb, v1r, dvr, w1r, xr, yr, xnr, ynr):
    z = dvr[...] * (qa[...] + qb[...] + v1r[...])
    xnr[...] = xr[...] + jnp.dot(z, w1r[DH:].T, preferred_element_type=jnp.float32)
    ynr[...] = yr[...] - jnp.dot(z, w1r[:DH].T, preferred_element_type=jnp.float32)


def _dec_body(xr, wr, br, outr):
    outr[...] = jnp.dot(xr[...], wr[...], preferred_element_type=jnp.float32) + br[...]


def kernel(x, edge_index, W_enc, b_enc, W1, b1, W2, b2, W3, b3, W_dec, b_dec):
    f32 = jnp.float32
    src = edge_index[0]
    dst = edge_index[1]
    z128 = jnp.zeros((RPT, 128), f32)
    ones128 = jnp.ones((N, 128), f32)

    npad = EPAD - E
    ppt = npad // NW
    ept = E // NW
    gpad = (jnp.arange(npad, dtype=jnp.int32) % N).reshape(NW, ppt)
    spad = (N + jnp.arange(npad, dtype=jnp.int32) % (NP - N)).reshape(NW, ppt)

    def _tile_layout(real, pad):
        return jnp.concatenate([real.reshape(NW, ept), pad], axis=1).reshape(-1)

    g_fwd = _tile_layout(src, gpad)
    s_fwd = _tile_layout(dst, spad)
    g_bwd = _tile_layout(dst, gpad)
    s_bwd = _tile_layout(src, spad)

    degp = _prop128(ones128, g_fwd, s_fwd, z128)
    deg = degp[:N, 0] + degp[NP:NP + N, 0] + 1.0
    dinv = lax.rsqrt(deg)
    dinv128 = jnp.broadcast_to(dinv[:, None], (N, 128))
    ctp = _prop128(dinv128, g_bwd, s_bwd, z128)
    c0 = dinv * (ctp[:N, 0] + ctp[NP:NP + N, 0] + dinv)

    dv2 = dinv[:, None]
    c2 = c0[:, None]
    b1_ = b1[None, :]
    b2p = jnp.concatenate([b2, jnp.zeros((DH,), f32)])[None, :]
    benc_ = b_enc[None, :]
    bdec_ = b_dec[None, :]
    W2p = jnp.concatenate([W2, jnp.zeros((128, DH), f32)], axis=1)
    w3p = jnp.concatenate([W3[:, 0], jnp.zeros((DH,), f32)])[None, :]

    sc_dv = pl.BlockSpec((ROW_BLK, 1), lambda i: (i, 0))

    Y = _tc_call(
        _enc_body,
        [_row_spec(128), _full_spec(128, DH), _full_spec(1, DH)],
        jax.ShapeDtypeStruct((N, DH), f32),
        _row_spec(DH),
    )(x, W_enc, benc_)
    X = Y

    stage1 = _tc_call(
        _stage1_body,
        [_row_spec(DH), _row_spec(DH), _full_spec(128, 128), sc_dv],
        jax.ShapeDtypeStruct((N, 128), f32),
        _row_spec(128),
    )
    stage2 = _tc_call(
        _stage2_body,
        [_row_spec(128), _row_spec(128), _row_spec(128), sc_dv,
         _full_spec(1, 128), _full_spec(128, 128)],
        [jax.ShapeDtypeStruct((N, 128), f32), jax.ShapeDtypeStruct((N, 128), f32)],
        [_row_spec(128), _row_spec(128)],
    )
    stage3 = _tc_call(
        _stage3_body,
        [_row_spec(128), _row_spec(128), _row_spec(128), sc_dv,
         _full_spec(1, 128), sc_dv, _full_spec(1, 128)],
        jax.ShapeDtypeStruct((N, 128), f32),
        _row_spec(128),
    )
    stage4 = _tc_call(
        _stage4_body,
        [_row_spec(128), _row_spec(128), _row_spec(128), sc_dv,
         _row_spec(128), _full_spec(128, 128)],
        jax.ShapeDtypeStruct((N, 128), f32),
        _row_spec(128),
    )
    stage5 = _tc_call(
        _stage5_body,
        [_row_spec(128), _row_spec(128), _row_spec(128), sc_dv,
         _full_spec(128, 128), _row_spec(DH), _row_spec(DH)],
        [jax.ShapeDtypeStruct((N, DH), f32), jax.ShapeDtypeStruct((N, DH), f32)],
        [_row_spec(DH), _row_spec(DH)],
    )

    for _ in range(2):
        u0 = stage1(X, Y, W1, dv2)
        p0 = _prop128(u0, g_fwd, s_fwd, z128)
        o1, u1 = stage2(p0[:N], p0[NP:NP + N], u0, dv2, b1_, W2p)
        p1 = _prop128(u1, g_fwd, s_fwd, z128)
        v2 = stage3(p1[:N], p1[NP:NP + N], u1, dv2, b2p, c2, w3p)
        q2 = _prop128(v2, g_bwd, s_bwd, z128)
        v1 = stage4(q2[:N], q2[NP:NP + N], v2, dv2, o1, W2p)
        q1 = _prop128(v1, g_bwd, s_bwd, z128)
        X, Y = stage5(q1[:N], q1[NP:NP + N], v1, dv2, W1, X, Y)

    out = _tc_call(
        _dec_body,
        [_row_spec(DH), _full_spec(DH, 16), _full_spec(1, 16)],
        jax.ShapeDtypeStruct((N, 16), f32),
        _row_spec(16),
    )(X, W_dec, bdec_)
    return out


# speedup vs baseline: 4.1039x; 1.9337x over previous
import functools

import jax
import jax.numpy as jnp
from jax import lax
from jax.experimental import pallas as pl
from jax.experimental.pallas import tpu as pltpu
from jax.experimental.pallas import tpu_sc as plsc

N = 10000
E = 320000
DH = 64
NC = 2
NS = 16
NW = NC * NS
KCH = 80
NCHUNK = 128
EPAD = NW * NCHUNK * KCH
NP = N + 1008
RPT = NP // NS

ROW_BLK = 1000
GRID = N // ROW_BLK


def _make_prop(d):
    mesh = plsc.VectorSubcoreMesh(
        core_axis_name="c", subcore_axis_name="s", num_cores=NC, num_subcores=NS
    )

    @functools.partial(
        pl.kernel,
        out_type=jax.ShapeDtypeStruct((NC * NP, d), jnp.float32),
        mesh=mesh,
        scratch_types=[
            pltpu.VMEM_SHARED((NP, d), jnp.float32),
            pltpu.VMEM((KCH,), jnp.int32),
            pltpu.VMEM((KCH,), jnp.int32),
            pltpu.VMEM((KCH,), jnp.int32),
            pltpu.VMEM((KCH,), jnp.int32),
            pltpu.VMEM((KCH, d), jnp.float32),
            pltpu.VMEM((KCH, d), jnp.float32),
            pltpu.SemaphoreType.DMA,
            pltpu.SemaphoreType.DMA,
            pltpu.SemaphoreType.DMA,
            pltpu.SemaphoreType.DMA,
        ],
    )
    def prop(table, idxg, idxs, zeros, out, acc, g0, s0, g1, s1, rows_a, rows_b,
             si0, si1, sa, sb):
        cid = lax.axis_index("c")
        sid = lax.axis_index("s")
        wid = cid * NS + sid
        r0 = sid * RPT
        pltpu.sync_copy(zeros, acc.at[pl.ds(r0, RPT)])
        plsc.subcore_barrier()

        e0 = wid * NCHUNK * KCH
        ganysrc = idxg.at[pl.ds(0, KCH)]
        tanysrc = table.at[g0]

        def _widx(buf, sem):
            pltpu.make_async_copy(ganysrc, buf, sem).wait()

        def _fire_idx(i, gbuf, sbuf, sem):
            off = pl.multiple_of(e0 + i * KCH, 8)
            pltpu.async_copy(idxg.at[pl.ds(off, KCH)], gbuf, sem)
            pltpu.async_copy(idxs.at[pl.ds(off, KCH)], sbuf, sem)

        _fire_idx(0, g0, s0, si0)
        _fire_idx(1, g1, s1, si1)
        _widx(g0, si0)
        _widx(s0, si0)
        pltpu.async_copy(table.at[g0], rows_a, sa)

        def body(j, carry):
            _widx(g1, si1)
            _widx(s1, si1)
            pltpu.async_copy(table.at[g1], rows_b, sb)
            pltpu.make_async_copy(tanysrc, rows_a, sa).wait()
            pltpu.sync_copy(rows_a, acc.at[s0], add=True)
            _fire_idx(2 * j + 2, g0, s0, si0)
            pltpu.make_async_copy(tanysrc, rows_b, sb).wait()
            pltpu.sync_copy(rows_b, acc.at[s1], add=True)
            _fire_idx(2 * j + 3, g1, s1, si1)
            _widx(g0, si0)
            _widx(s0, si0)
            pltpu.async_copy(table.at[g0], rows_a, sa)
            return carry

        lax.fori_loop(0, NCHUNK // 2 - 1, body, 0)

        _widx(g1, si1)
        _widx(s1, si1)
        pltpu.async_copy(table.at[g1], rows_b, sb)
        pltpu.make_async_copy(tanysrc, rows_a, sa).wait()
        pltpu.sync_copy(rows_a, acc.at[s0], add=True)
        pltpu.make_async_copy(tanysrc, rows_b, sb).wait()
        pltpu.sync_copy(rows_b, acc.at[s1], add=True)
        plsc.subcore_barrier()
        pltpu.sync_copy(
            acc.at[pl.ds(r0, RPT)], out.at[pl.ds(cid * NP + r0, RPT)]
        )

    return prop


_prop128 = _make_prop(128)


def _row_spec(cols):
    return pl.BlockSpec((ROW_BLK, cols), lambda i: (i, 0))


def _pair_spec(cols):
    return pl.BlockSpec((ROW_BLK, cols), lambda i: (i, 0))


def _full_spec(rows, cols):
    return pl.BlockSpec((rows, cols), lambda i: (0, 0))


def _tc_call(body, in_specs, out_shape, out_specs):
    return pl.pallas_call(
        body,
        grid=(GRID,),
        in_specs=in_specs,
        out_shape=out_shape,
        out_specs=out_specs,
    )


def _enc_body(x_ref, w_ref, b_ref, y_ref):
    y = jnp.dot(x_ref[...], w_ref[...], preferred_element_type=jnp.float32)
    y_ref[...] = jnp.maximum(y + b_ref[...], 0.0)


def _stage1_body(xr, yr, w1r, dvr, ur):
    acc = jnp.dot(xr[...], w1r[:DH], preferred_element_type=jnp.float32)
    acc += jnp.dot(yr[...], w1r[DH:], preferred_element_type=jnp.float32)
    ur[...] = dvr[...] * acc


def _stage2_body(pa, pb, ur, dvr, b1r, w2pr, o1r, u1r):
    o1 = jnp.tanh(dvr[...] * (pa[...] + pb[...] + ur[...]) + b1r[...])
    o1r[...] = o1
    u1r[...] = dvr[...] * jnp.dot(o1, w2pr[...], preferred_element_type=jnp.float32)


def _stage3_body(pa, pb, u1r, dvr, b2r, cr, w3pr, v2r):
    o2 = jnp.tanh(dvr[...] * (pa[...] + pb[...] + u1r[...]) + b2r[...])
    v2r[...] = dvr[...] * (1.0 - o2 * o2) * (cr[...] * w3pr[...])


def _stage4_body(qa, qb, v2r, dvr, o1r, w2pr, v1r):
    t = dvr[...] * (qa[...] + qb[...] + v2r[...])
    go1 = jnp.dot(t, w2pr[...].T, preferred_element_type=jnp.float32)
    o1 = o1r[...]
    v1r[...] = dvr[...] * (1.0 - o1 * o1) * go1


def _stage5_body(qa, qb, v1r, dvr, w1r, xr, yr, xnr, ynr):
    z = dvr[...] * (qa[...] + qb[...] + v1r[...])
    xnr[...] = xr[...] + jnp.dot(z, w1r[DH:].T, preferred_element_type=jnp.float32)
    ynr[...] = yr[...] - jnp.dot(z, w1r[:DH].T, preferred_element_type=jnp.float32)


def _dec_body(xr, wr, br, outr):
    outr[...] = jnp.dot(xr[...], wr[...], preferred_element_type=jnp.float32) + br[...]


def kernel(x, edge_index, W_enc, b_enc, W1, b1, W2, b2, W3, b3, W_dec, b_dec):
    f32 = jnp.float32
    src = edge_index[0]
    dst = edge_index[1]
    z128 = jnp.zeros((RPT, 128), f32)
    ones128 = jnp.ones((N, 128), f32)

    npad = EPAD - E
    ppt = npad // NW
    ept = E // NW
    gpad = (jnp.arange(npad, dtype=jnp.int32) % N).reshape(NW, ppt)
    spad = (N + jnp.arange(npad, dtype=jnp.int32) % (NP - N)).reshape(NW, ppt)

    def _tile_layout(real, pad):
        return jnp.concatenate([real.reshape(NW, ept), pad], axis=1).reshape(-1)

    g_fwd = _tile_layout(src, gpad)
    s_fwd = _tile_layout(dst, spad)
    g_bwd = _tile_layout(dst, gpad)
    s_bwd = _tile_layout(src, spad)

    degp = _prop128(ones128, g_fwd, s_fwd, z128)
    deg = degp[:N, 0] + degp[NP:NP + N, 0] + 1.0
    dinv = lax.rsqrt(deg)
    dinv128 = jnp.broadcast_to(dinv[:, None], (N, 128))
    ctp = _prop128(dinv128, g_bwd, s_bwd, z128)
    c0 = dinv * (ctp[:N, 0] + ctp[NP:NP + N, 0] + dinv)

    dv2 = dinv[:, None]
    c2 = c0[:, None]
    b1_ = b1[None, :]
    b2p = jnp.concatenate([b2, jnp.zeros((DH,), f32)])[None, :]
    benc_ = b_enc[None, :]
    bdec_ = b_dec[None, :]
    W2p = jnp.concatenate([W2, jnp.zeros((128, DH), f32)], axis=1)
    w3p = jnp.concatenate([W3[:, 0], jnp.zeros((DH,), f32)])[None, :]

    sc_dv = pl.BlockSpec((ROW_BLK, 1), lambda i: (i, 0))

    Y = _tc_call(
        _enc_body,
        [_row_spec(128), _full_spec(128, DH), _full_spec(1, DH)],
        jax.ShapeDtypeStruct((N, DH), f32),
        _row_spec(DH),
    )(x, W_enc, benc_)
    X = Y

    stage1 = _tc_call(
        _stage1_body,
        [_row_spec(DH), _row_spec(DH), _full_spec(128, 128), sc_dv],
        jax.ShapeDtypeStruct((N, 128), f32),
        _row_spec(128),
    )
    stage2 = _tc_call(
        _stage2_body,
        [_row_spec(128), _row_spec(128), _row_spec(128), sc_dv,
         _full_spec(1, 128), _full_spec(128, 128)],
        [jax.ShapeDtypeStruct((N, 128), f32), jax.ShapeDtypeStruct((N, 128), f32)],
        [_row_spec(128), _row_spec(128)],
    )
    stage3 = _tc_call(
        _stage3_body,
        [_row_spec(128), _row_spec(128), _row_spec(128), sc_dv,
         _full_spec(1, 128), sc_dv, _full_spec(1, 128)],
        jax.ShapeDtypeStruct((N, 128), f32),
        _row_spec(128),
    )
    stage4 = _tc_call(
        _stage4_body,
        [_row_spec(128), _row_spec(128), _row_spec(128), sc_dv,
         _row_spec(128), _full_spec(128, 128)],
        jax.ShapeDtypeStruct((N, 128), f32),
        _row_spec(128),
    )
    stage5 = _tc_call(
        _stage5_body,
        [_row_spec(128), _row_spec(128), _row_spec(128), sc_dv,
         _full_spec(128, 128), _row_spec(DH), _row_spec(DH)],
        [jax.ShapeDtypeStruct((N, DH), f32), jax.ShapeDtypeStruct((N, DH), f32)],
        [_row_spec(DH), _row_spec(DH)],
    )

    for _ in range(2):
        u0 = stage1(X, Y, W1, dv2)
        p0 = _prop128(u0, g_fwd, s_fwd, z128)
        o1, u1 = stage2(p0[:N], p0[NP:NP + N], u0, dv2, b1_, W2p)
        p1 = _prop128(u1, g_fwd, s_fwd, z128)
        v2 = stage3(p1[:N], p1[NP:NP + N], u1, dv2, b2p, c2, w3p)
        q2 = _prop128(v2, g_bwd, s_bwd, z128)
        v1 = stage4(q2[:N], q2[NP:NP + N], v2, dv2, o1, W2p)
        q1 = _prop128(v1, g_bwd, s_bwd, z128)
        X, Y = stage5(q1[:N], q1[NP:NP + N], v1, dv2, W1, X, Y)

    out = _tc_call(
        _dec_body,
        [_row_spec(DH), _full_spec(DH, 16), _full_spec(1, 16)],
        jax.ShapeDtypeStruct((N, 16), f32),
        _row_spec(16),
    )(X, W_dec, bdec_)
    return out


# speedup vs baseline: 4.3794x; 1.0671x over previous
import functools

import jax
import jax.numpy as jnp
from jax import lax
from jax.experimental import pallas as pl
from jax.experimental.pallas import tpu as pltpu
from jax.experimental.pallas import tpu_sc as plsc

N = 10000
E = 320000
DH = 64
NC = 2
NS = 16
NW = NC * NS
KCH = 128
NCHUNK = 80
EPAD = NW * NCHUNK * KCH
NP = N + 1008
RPT = NP // NS

ROW_BLK = 1000
GRID = N // ROW_BLK


def _make_prop(d):
    mesh = plsc.VectorSubcoreMesh(
        core_axis_name="c", subcore_axis_name="s", num_cores=NC, num_subcores=NS
    )

    @functools.partial(
        pl.kernel,
        out_type=jax.ShapeDtypeStruct((NC * NP, d), jnp.float32),
        mesh=mesh,
        scratch_types=[
            pltpu.VMEM_SHARED((NP, d), jnp.float32),
            pltpu.VMEM((KCH,), jnp.int32),
            pltpu.VMEM((KCH,), jnp.int32),
            pltpu.VMEM((KCH,), jnp.int32),
            pltpu.VMEM((KCH,), jnp.int32),
            pltpu.VMEM((KCH, d), jnp.float32),
            pltpu.VMEM((KCH, d), jnp.float32),
            pltpu.SemaphoreType.DMA,
            pltpu.SemaphoreType.DMA,
            pltpu.SemaphoreType.DMA,
            pltpu.SemaphoreType.DMA,
        ],
    )
    def prop(table, idxg, idxs, zeros, out, acc, g0, s0, g1, s1, rows_a, rows_b,
             si0, si1, sa, sb):
        cid = lax.axis_index("c")
        sid = lax.axis_index("s")
        wid = cid * NS + sid
        r0 = sid * RPT
        pltpu.sync_copy(zeros, acc.at[pl.ds(r0, RPT)])
        plsc.subcore_barrier()

        e0 = wid * NCHUNK * KCH
        ganysrc = idxg.at[pl.ds(0, KCH)]
        tanysrc = table.at[g0]

        def _widx(buf, sem):
            pltpu.make_async_copy(ganysrc, buf, sem).wait()

        def _fire_idx(i, gbuf, sbuf, sem):
            off = pl.multiple_of(e0 + i * KCH, 8)
            pltpu.async_copy(idxg.at[pl.ds(off, KCH)], gbuf, sem)
            pltpu.async_copy(idxs.at[pl.ds(off, KCH)], sbuf, sem)

        _fire_idx(0, g0, s0, si0)
        _fire_idx(1, g1, s1, si1)
        _widx(g0, si0)
        _widx(s0, si0)
        pltpu.async_copy(table.at[g0], rows_a, sa)

        def body(j, carry):
            _widx(g1, si1)
            _widx(s1, si1)
            pltpu.async_copy(table.at[g1], rows_b, sb)
            pltpu.make_async_copy(tanysrc, rows_a, sa).wait()
            pltpu.sync_copy(rows_a, acc.at[s0], add=True)
            _fire_idx(2 * j + 2, g0, s0, si0)
            pltpu.make_async_copy(tanysrc, rows_b, sb).wait()
            pltpu.sync_copy(rows_b, acc.at[s1], add=True)
            _fire_idx(2 * j + 3, g1, s1, si1)
            _widx(g0, si0)
            _widx(s0, si0)
            pltpu.async_copy(table.at[g0], rows_a, sa)
            return carry

        lax.fori_loop(0, NCHUNK // 2 - 1, body, 0)

        _widx(g1, si1)
        _widx(s1, si1)
        pltpu.async_copy(table.at[g1], rows_b, sb)
        pltpu.make_async_copy(tanysrc, rows_a, sa).wait()
        pltpu.sync_copy(rows_a, acc.at[s0], add=True)
        pltpu.make_async_copy(tanysrc, rows_b, sb).wait()
        pltpu.sync_copy(rows_b, acc.at[s1], add=True)
        plsc.subcore_barrier()
        pltpu.sync_copy(
            acc.at[pl.ds(r0, RPT)], out.at[pl.ds(cid * NP + r0, RPT)]
        )

    return prop


_prop128 = _make_prop(128)


def _row_spec(cols):
    return pl.BlockSpec((ROW_BLK, cols), lambda i: (i, 0))


def _pair_spec(cols):
    return pl.BlockSpec((ROW_BLK, cols), lambda i: (i, 0))


def _full_spec(rows, cols):
    return pl.BlockSpec((rows, cols), lambda i: (0, 0))


def _tc_call(body, in_specs, out_shape, out_specs):
    return pl.pallas_call(
        body,
        grid=(GRID,),
        in_specs=in_specs,
        out_shape=out_shape,
        out_specs=out_specs,
    )


def _enc_body(x_ref, w_ref, b_ref, y_ref):
    y = jnp.dot(x_ref[...], w_ref[...], preferred_element_type=jnp.float32)
    y_ref[...] = jnp.maximum(y + b_ref[...], 0.0)


def _stage1_body(xr, yr, w1r, dvr, ur):
    acc = jnp.dot(xr[...], w1r[:DH], preferred_element_type=jnp.float32)
    acc += jnp.dot(yr[...], w1r[DH:], preferred_element_type=jnp.float32)
    ur[...] = dvr[...] * acc


def _stage2_body(pa, pb, ur, dvr, b1r, w2pr, o1r, u1r):
    o1 = jnp.tanh(dvr[...] * (pa[...] + pb[...] + ur[...]) + b1r[...])
    o1r[...] = o1
    u1r[...] = dvr[...] * jnp.dot(o1, w2pr[...], preferred_element_type=jnp.float32)


def _stage3_body(pa, pb, u1r, dvr, b2r, cr, w3pr, v2r):
    o2 = jnp.tanh(dvr[...] * (pa[...] + pb[...] + u1r[...]) + b2r[...])
    v2r[...] = dvr[...] * (1.0 - o2 * o2) * (cr[...] * w3pr[...])


def _stage4_body(qa, qb, v2r, dvr, o1r, w2pr, v1r):
    t = dvr[...] * (qa[...] + qb[...] + v2r[...])
    go1 = jnp.dot(t, w2pr[...].T, preferred_element_type=jnp.float32)
    o1 = o1r[...]
    v1r[...] = dvr[...] * (1.0 - o1 * o1) * go1


def _stage5_body(qa, qb, v1r, dvr, w1r, xr, yr, xnr, ynr):
    z = dvr[...] * (qa[...] + qb[...] + v1r[...])
    xnr[...] = xr[...] + jnp.dot(z, w1r[DH:].T, preferred_element_type=jnp.float32)
    ynr[...] = yr[...] - jnp.dot(z, w1r[:DH].T, preferred_element_type=jnp.float32)


def _dec_body(xr, wr, br, outr):
    outr[...] = jnp.dot(xr[...], wr[...], preferred_element_type=jnp.float32) + br[...]


def kernel(x, edge_index, W_enc, b_enc, W1, b1, W2, b2, W3, b3, W_dec, b_dec):
    f32 = jnp.float32
    src = edge_index[0]
    dst = edge_index[1]
    z128 = jnp.zeros((RPT, 128), f32)
    ones128 = jnp.ones((N, 128), f32)

    npad = EPAD - E
    ppt = npad // NW
    ept = E // NW
    gpad = (jnp.arange(npad, dtype=jnp.int32) % N).reshape(NW, ppt)
    spad = (N + jnp.arange(npad, dtype=jnp.int32) % (NP - N)).reshape(NW, ppt)

    def _tile_layout(real, pad):
        return jnp.concatenate([real.reshape(NW, ept), pad], axis=1).reshape(-1)

    g_fwd = _tile_layout(src, gpad)
    s_fwd = _tile_layout(dst, spad)
    g_bwd = _tile_layout(dst, gpad)
    s_bwd = _tile_layout(src, spad)

    degp = _prop128(ones128, g_fwd, s_fwd, z128)
    deg = degp[:N, 0] + degp[NP:NP + N, 0] + 1.0
    dinv = lax.rsqrt(deg)
    dinv128 = jnp.broadcast_to(dinv[:, None], (N, 128))
    ctp = _prop128(dinv128, g_bwd, s_bwd, z128)
    c0 = dinv * (ctp[:N, 0] + ctp[NP:NP + N, 0] + dinv)

    dv2 = dinv[:, None]
    c2 = c0[:, None]
    b1_ = b1[None, :]
    b2p = jnp.concatenate([b2, jnp.zeros((DH,), f32)])[None, :]
    benc_ = b_enc[None, :]
    bdec_ = b_dec[None, :]
    W2p = jnp.concatenate([W2, jnp.zeros((128, DH), f32)], axis=1)
    w3p = jnp.concatenate([W3[:, 0], jnp.zeros((DH,), f32)])[None, :]

    sc_dv = pl.BlockSpec((ROW_BLK, 1), lambda i: (i, 0))

    Y = _tc_call(
        _enc_body,
        [_row_spec(128), _full_spec(128, DH), _full_spec(1, DH)],
        jax.ShapeDtypeStruct((N, DH), f32),
        _row_spec(DH),
    )(x, W_enc, benc_)
    X = Y

    stage1 = _tc_call(
        _stage1_body,
        [_row_spec(DH), _row_spec(DH), _full_spec(128, 128), sc_dv],
        jax.ShapeDtypeStruct((N, 128), f32),
        _row_spec(128),
    )
    stage2 = _tc_call(
        _stage2_body,
        [_row_spec(128), _row_spec(128), _row_spec(128), sc_dv,
         _full_spec(1, 128), _full_spec(128, 128)],
        [jax.ShapeDtypeStruct((N, 128), f32), jax.ShapeDtypeStruct((N, 128), f32)],
        [_row_spec(128), _row_spec(128)],
    )
    stage3 = _tc_call(
        _stage3_body,
        [_row_spec(128), _row_spec(128), _row_spec(128), sc_dv,
         _full_spec(1, 128), sc_dv, _full_spec(1, 128)],
        jax.ShapeDtypeStruct((N, 128), f32),
        _row_spec(128),
    )
    stage4 = _tc_call(
        _stage4_body,
        [_row_spec(128), _row_spec(128), _row_spec(128), sc_dv,
         _row_spec(128), _full_spec(128, 128)],
        jax.ShapeDtypeStruct((N, 128), f32),
        _row_spec(128),
    )
    stage5 = _tc_call(
        _stage5_body,
        [_row_spec(128), _row_spec(128), _row_spec(128), sc_dv,
         _full_spec(128, 128), _row_spec(DH), _row_spec(DH)],
        [jax.ShapeDtypeStruct((N, DH), f32), jax.ShapeDtypeStruct((N, DH), f32)],
        [_row_spec(DH), _row_spec(DH)],
    )

    for _ in range(2):
        u0 = stage1(X, Y, W1, dv2)
        p0 = _prop128(u0, g_fwd, s_fwd, z128)
        o1, u1 = stage2(p0[:N], p0[NP:NP + N], u0, dv2, b1_, W2p)
        p1 = _prop128(u1, g_fwd, s_fwd, z128)
        v2 = stage3(p1[:N], p1[NP:NP + N], u1, dv2, b2p, c2, w3p)
        q2 = _prop128(v2, g_bwd, s_bwd, z128)
        v1 = stage4(q2[:N], q2[NP:NP + N], v2, dv2, o1, W2p)
        q1 = _prop128(v1, g_bwd, s_bwd, z128)
        X, Y = stage5(q1[:N], q1[NP:NP + N], v1, dv2, W1, X, Y)

    out = _tc_call(
        _dec_body,
        [_row_spec(DH), _full_spec(DH, 16), _full_spec(1, 16)],
        jax.ShapeDtypeStruct((N, 16), f32),
        _row_spec(16),
    )(X, W_dec, bdec_)
    return out


# speedup vs baseline: 4.5722x; 1.0440x over previous
import functools

import jax
import jax.numpy as jnp
from jax import lax
from jax.experimental import pallas as pl
from jax.experimental.pallas import tpu as pltpu
from jax.experimental.pallas import tpu_sc as plsc

N = 10000
E = 320000
DH = 64
NC = 2
NS = 16
NW = NC * NS
KCH = 128
NCHUNK = 80
PH = 2
CPP = NCHUNK // PH
NH = CPP // 2
EPAD = NW * NCHUNK * KCH
TOTCH = EPAD // KCH
NP = N + 496
RPT = NP // NS

ROW_BLK = 1000
GRID = N // ROW_BLK


def _make_prop(d):
    mesh = plsc.VectorSubcoreMesh(
        core_axis_name="c", subcore_axis_name="s", num_cores=NC, num_subcores=NS
    )

    @functools.partial(
        pl.kernel,
        out_type=jax.ShapeDtypeStruct((NC * NP, d), jnp.float32),
        mesh=mesh,
        scratch_types=[
            pltpu.VMEM_SHARED((NP, d), jnp.float32),
            pltpu.VMEM((CPP, KCH), jnp.int32),
            pltpu.VMEM((CPP, KCH), jnp.int32),
            pltpu.VMEM((KCH, d), jnp.float32),
            pltpu.VMEM((KCH, d), jnp.float32),
            pltpu.SemaphoreType.DMA,
            pltpu.SemaphoreType.DMA,
            pltpu.SemaphoreType.DMA,
            pltpu.SemaphoreType.DMA,
        ],
    )
    def prop(table, idxg, idxs, zeros, out, acc, gidx_v, sidx_v, rows_a, rows_b,
             sa, sb, pa, pb):
        cid = lax.axis_index("c")
        sid = lax.axis_index("s")
        wid = cid * NS + sid
        r0 = sid * RPT
        pltpu.sync_copy(zeros, acc.at[pl.ds(r0, RPT)])
        plsc.subcore_barrier()

        c0 = wid * NCHUNK

        def _fire_g(i, rows, sem):
            pltpu.async_copy(table.at[gidx_v.at[i]], rows, sem)

        def _wait_g(i, rows, sem):
            pltpu.make_async_copy(table.at[gidx_v.at[i]], rows, sem).wait()

        def _fire_s(i, rows, sem):
            pltpu.async_copy(rows, acc.at[sidx_v.at[i]], sem, add=True)

        def _wait_s(i, rows, sem):
            pltpu.make_async_copy(rows, acc.at[sidx_v.at[i]], sem).wait()

        for p in range(PH):
            pltpu.sync_copy(idxg.at[pl.ds(c0 + p * CPP, CPP)], gidx_v)
            pltpu.sync_copy(idxs.at[pl.ds(c0 + p * CPP, CPP)], sidx_v)
            _fire_g(0, rows_a, sa)
            _wait_g(0, rows_a, sa)
            _fire_s(0, rows_a, pa)
            _fire_g(1, rows_b, sb)
            _wait_g(1, rows_b, sb)
            _fire_s(1, rows_b, pb)
            _wait_s(0, rows_a, pa)
            _fire_g(2, rows_a, sa)

            def body(j, carry):
                i0 = 2 * j
                i1 = i0 + 1
                _wait_g(i0, rows_a, sa)
                _fire_s(i0, rows_a, pa)
                _wait_s(i1 - 2, rows_b, pb)
                _fire_g(i1, rows_b, sb)
                _wait_g(i1, rows_b, sb)
                _fire_s(i1, rows_b, pb)
                _wait_s(i0, rows_a, pa)
                _fire_g(i0 + 2, rows_a, sa)
                return carry

            lax.fori_loop(1, NH - 1, body, 0)

            i0 = CPP - 2
            i1 = CPP - 1
            _wait_g(i0, rows_a, sa)
            _fire_s(i0, rows_a, pa)
            _wait_s(i1 - 2, rows_b, pb)
            _fire_g(i1, rows_b, sb)
            _wait_g(i1, rows_b, sb)
            _fire_s(i1, rows_b, pb)
            _wait_s(i0, rows_a, pa)
            _wait_s(i1, rows_b, pb)

        plsc.subcore_barrier()
        pltpu.sync_copy(
            acc.at[pl.ds(r0, RPT)], out.at[pl.ds(cid * NP + r0, RPT)]
        )

    return prop


_prop128 = _make_prop(128)


def _row_spec(cols):
    return pl.BlockSpec((ROW_BLK, cols), lambda i: (i, 0))


def _pair_spec(cols):
    return pl.BlockSpec((ROW_BLK, cols), lambda i: (i, 0))


def _full_spec(rows, cols):
    return pl.BlockSpec((rows, cols), lambda i: (0, 0))


def _tc_call(body, in_specs, out_shape, out_specs):
    return pl.pallas_call(
        body,
        grid=(GRID,),
        in_specs=in_specs,
        out_shape=out_shape,
        out_specs=out_specs,
    )


def _enc_body(x_ref, w_ref, b_ref, y_ref):
    y = jnp.dot(x_ref[...], w_ref[...], preferred_element_type=jnp.float32)
    y_ref[...] = jnp.maximum(y + b_ref[...], 0.0)


def _stage1_body(xr, yr, w1r, dvr, ur):
    acc = jnp.dot(xr[...], w1r[:DH], preferred_element_type=jnp.float32)
    acc += jnp.dot(yr[...], w1r[DH:], preferred_element_type=jnp.float32)
    ur[...] = dvr[...] * acc


def _stage2_body(pa, pb, ur, dvr, b1r, w2pr, o1r, u1r):
    o1 = jnp.tanh(dvr[...] * (pa[...] + pb[...] + ur[...]) + b1r[...])
    o1r[...] = o1
    u1r[...] = dvr[...] * jnp.dot(o1, w2pr[...], preferred_element_type=jnp.float32)


def _stage3_body(pa, pb, u1r, dvr, b2r, cr, w3pr, v2r):
    o2 = jnp.tanh(dvr[...] * (pa[...] + pb[...] + u1r[...]) + b2r[...])
    v2r[...] = dvr[...] * (1.0 - o2 * o2) * (cr[...] * w3pr[...])


def _stage4_body(qa, qb, v2r, dvr, o1r, w2pr, v1r):
    t = dvr[...] * (qa[...] + qb[...] + v2r[...])
    go1 = jnp.dot(t, w2pr[...].T, preferred_element_type=jnp.float32)
    o1 = o1r[...]
    v1r[...] = dvr[...] * (1.0 - o1 * o1) * go1


def _stage5_body(qa, qb, v1r, dvr, w1r, xr, yr, xnr, ynr):
    z = dvr[...] * (qa[...] + qb[...] + v1r[...])
    xnr[...] = xr[...] + jnp.dot(z, w1r[DH:].T, preferred_element_type=jnp.float32)
    ynr[...] = yr[...] - jnp.dot(z, w1r[:DH].T, preferred_element_type=jnp.float32)


def _dec_body(xr, wr, br, outr):
    outr[...] = jnp.dot(xr[...], wr[...], preferred_element_type=jnp.float32) + br[...]


def kernel(x, edge_index, W_enc, b_enc, W1, b1, W2, b2, W3, b3, W_dec, b_dec):
    f32 = jnp.float32
    src = edge_index[0]
    dst = edge_index[1]
    z128 = jnp.zeros((RPT, 128), f32)
    ones128 = jnp.ones((N, 128), f32)

    npad = EPAD - E
    ppt = npad // NW
    ept = E // NW
    gpad = (jnp.arange(npad, dtype=jnp.int32) % N).reshape(NW, ppt)
    spad = (N + jnp.arange(npad, dtype=jnp.int32) % (NP - N)).reshape(NW, ppt)

    def _tile_layout(real, pad):
        return jnp.concatenate([real.reshape(NW, ept), pad], axis=1).reshape(-1)

    g_fwd = _tile_layout(src, gpad).reshape(TOTCH, KCH)
    s_fwd = _tile_layout(dst, spad).reshape(TOTCH, KCH)
    g_bwd = _tile_layout(dst, gpad).reshape(TOTCH, KCH)
    s_bwd = _tile_layout(src, spad).reshape(TOTCH, KCH)

    degp = _prop128(ones128, g_fwd, s_fwd, z128)
    deg = degp[:N, 0] + degp[NP:NP + N, 0] + 1.0
    dinv = lax.rsqrt(deg)
    dinv128 = jnp.broadcast_to(dinv[:, None], (N, 128))
    ctp = _prop128(dinv128, g_bwd, s_bwd, z128)
    c0 = dinv * (ctp[:N, 0] + ctp[NP:NP + N, 0] + dinv)

    dv2 = dinv[:, None]
    c2 = c0[:, None]
    b1_ = b1[None, :]
    b2p = jnp.concatenate([b2, jnp.zeros((DH,), f32)])[None, :]
    benc_ = b_enc[None, :]
    bdec_ = b_dec[None, :]
    W2p = jnp.concatenate([W2, jnp.zeros((128, DH), f32)], axis=1)
    w3p = jnp.concatenate([W3[:, 0], jnp.zeros((DH,), f32)])[None, :]

    sc_dv = pl.BlockSpec((ROW_BLK, 1), lambda i: (i, 0))

    Y = _tc_call(
        _enc_body,
        [_row_spec(128), _full_spec(128, DH), _full_spec(1, DH)],
        jax.ShapeDtypeStruct((N, DH), f32),
        _row_spec(DH),
    )(x, W_enc, benc_)
    X = Y

    stage1 = _tc_call(
        _stage1_body,
        [_row_spec(DH), _row_spec(DH), _full_spec(128, 128), sc_dv],
        jax.ShapeDtypeStruct((N, 128), f32),
        _row_spec(128),
    )
    stage2 = _tc_call(
        _stage2_body,
        [_row_spec(128), _row_spec(128), _row_spec(128), sc_dv,
         _full_spec(1, 128), _full_spec(128, 128)],
        [jax.ShapeDtypeStruct((N, 128), f32), jax.ShapeDtypeStruct((N, 128), f32)],
        [_row_spec(128), _row_spec(128)],
    )
    stage3 = _tc_call(
        _stage3_body,
        [_row_spec(128), _row_spec(128), _row_spec(128), sc_dv,
         _full_spec(1, 128), sc_dv, _full_spec(1, 128)],
        jax.ShapeDtypeStruct((N, 128), f32),
        _row_spec(128),
    )
    stage4 = _tc_call(
        _stage4_body,
        [_row_spec(128), _row_spec(128), _row_spec(128), sc_dv,
         _row_spec(128), _full_spec(128, 128)],
        jax.ShapeDtypeStruct((N, 128), f32),
        _row_spec(128),
    )
    stage5 = _tc_call(
        _stage5_body,
        [_row_spec(128), _row_spec(128), _row_spec(128), sc_dv,
         _full_spec(128, 128), _row_spec(DH), _row_spec(DH)],
        [jax.ShapeDtypeStruct((N, DH), f32), jax.ShapeDtypeStruct((N, DH), f32)],
        [_row_spec(DH), _row_spec(DH)],
    )

    for _ in range(2):
        u0 = stage1(X, Y, W1, dv2)
        p0 = _prop128(u0, g_fwd, s_fwd, z128)
        o1, u1 = stage2(p0[:N], p0[NP:NP + N], u0, dv2, b1_, W2p)
        p1 = _prop128(u1, g_fwd, s_fwd, z128)
        v2 = stage3(p1[:N], p1[NP:NP + N], u1, dv2, b2p, c2, w3p)
        q2 = _prop128(v2, g_bwd, s_bwd, z128)
        v1 = stage4(q2[:N], q2[NP:NP + N], v2, dv2, o1, W2p)
        q1 = _prop128(v1, g_bwd, s_bwd, z128)
        X, Y = stage5(q1[:N], q1[NP:NP + N], v1, dv2, W1, X, Y)

    out = _tc_call(
        _dec_body,
        [_row_spec(DH), _full_spec(DH, 16), _full_spec(1, 16)],
        jax.ShapeDtypeStruct((N, 16), f32),
        _row_spec(16),
    )(X, W_dec, bdec_)
    return out


# speedup vs baseline: 4.7770x; 1.0448x over previous
import functools

import jax
import jax.numpy as jnp
from jax import lax
from jax.experimental import pallas as pl
from jax.experimental.pallas import tpu as pltpu
from jax.experimental.pallas import tpu_sc as plsc

N = 10000
E = 320000
DH = 64
NC = 2
NS = 16
NW = NC * NS
KCH = 128
NCHUNK = 80
PH = 2
CPP = NCHUNK // PH
NH = CPP // 2
EPAD = NW * NCHUNK * KCH
TOTCH = EPAD // KCH
NP = N + 496
RPT = NP // NS

ROW_BLK = 2000
GRID = N // ROW_BLK


def _make_prop(d, gather=True):
    mesh = plsc.VectorSubcoreMesh(
        core_axis_name="c", subcore_axis_name="s", num_cores=NC, num_subcores=NS
    )

    @functools.partial(
        pl.kernel,
        out_type=jax.ShapeDtypeStruct((NC * NP, d), jnp.float32),
        mesh=mesh,
        scratch_types=[
            pltpu.VMEM_SHARED((NP, d), jnp.float32),
            pltpu.VMEM((CPP, KCH), jnp.int32),
            pltpu.VMEM((CPP, KCH), jnp.int32),
            pltpu.VMEM((KCH, d), jnp.float32),
            pltpu.VMEM((KCH, d), jnp.float32),
            pltpu.SemaphoreType.DMA,
            pltpu.SemaphoreType.DMA,
            pltpu.SemaphoreType.DMA,
            pltpu.SemaphoreType.DMA,
        ],
    )
    def prop(table, idxg, idxs, zeros, out, acc, gidx_v, sidx_v, rows_a, rows_b,
             sa, sb, pa, pb):
        cid = lax.axis_index("c")
        sid = lax.axis_index("s")
        wid = cid * NS + sid
        r0 = sid * RPT
        pltpu.sync_copy(zeros, acc.at[pl.ds(r0, RPT)])
        plsc.subcore_barrier()

        c0 = wid * NCHUNK

        def _fire_g(i, rows, sem):
            pltpu.async_copy(table.at[gidx_v.at[i]], rows, sem)

        def _wait_g(i, rows, sem):
            pltpu.make_async_copy(table.at[gidx_v.at[i]], rows, sem).wait()

        def _fire_s(i, rows, sem):
            pltpu.async_copy(rows, acc.at[sidx_v.at[i]], sem, add=True)

        def _wait_s(i, rows, sem):
            pltpu.make_async_copy(rows, acc.at[sidx_v.at[i]], sem).wait()

        if not gather:
            pltpu.sync_copy(table.at[pl.ds(0, KCH)], rows_a)
            pltpu.sync_copy(table.at[pl.ds(0, KCH)], rows_b)

            def cbody(j, carry):
                pltpu.sync_copy(idxs.at[pl.ds(c0 + j, 1)], sidx_v.at[pl.ds(0, 1)])
                pltpu.sync_copy(rows_a, acc.at[sidx_v.at[0]], add=True)
                return carry

            for p in range(PH):
                pltpu.sync_copy(idxs.at[pl.ds(c0 + p * CPP, CPP)], sidx_v)

                def cbody2(j, carry):
                    pltpu.sync_copy(rows_a, acc.at[sidx_v.at[j]], add=True)
                    return carry

                lax.fori_loop(0, CPP, cbody2, 0)
            plsc.subcore_barrier()
            pltpu.sync_copy(
                acc.at[pl.ds(r0, RPT)], out.at[pl.ds(cid * NP + r0, RPT)]
            )
            return

        for p in range(PH):
            pltpu.sync_copy(idxg.at[pl.ds(c0 + p * CPP, CPP)], gidx_v)
            pltpu.sync_copy(idxs.at[pl.ds(c0 + p * CPP, CPP)], sidx_v)
            _fire_g(0, rows_a, sa)
            _wait_g(0, rows_a, sa)
            _fire_s(0, rows_a, pa)
            _fire_g(1, rows_b, sb)
            _wait_g(1, rows_b, sb)
            _fire_s(1, rows_b, pb)
            _wait_s(0, rows_a, pa)
            _fire_g(2, rows_a, sa)

            def body(j, carry):
                i0 = 2 * j
                i1 = i0 + 1
                _wait_g(i0, rows_a, sa)
                _fire_s(i0, rows_a, pa)
                _wait_s(i1 - 2, rows_b, pb)
                _fire_g(i1, rows_b, sb)
                _wait_g(i1, rows_b, sb)
                _fire_s(i1, rows_b, pb)
                _wait_s(i0, rows_a, pa)
                _fire_g(i0 + 2, rows_a, sa)
                return carry

            lax.fori_loop(1, NH - 1, body, 0)

            i0 = CPP - 2
            i1 = CPP - 1
            _wait_g(i0, rows_a, sa)
            _fire_s(i0, rows_a, pa)
            _wait_s(i1 - 2, rows_b, pb)
            _fire_g(i1, rows_b, sb)
            _wait_g(i1, rows_b, sb)
            _fire_s(i1, rows_b, pb)
            _wait_s(i0, rows_a, pa)
            _wait_s(i1, rows_b, pb)

        plsc.subcore_barrier()
        pltpu.sync_copy(
            acc.at[pl.ds(r0, RPT)], out.at[pl.ds(cid * NP + r0, RPT)]
        )

    return prop


_prop128 = _make_prop(128)
_count128 = _make_prop(128, gather=False)


def _row_spec(cols):
    return pl.BlockSpec((ROW_BLK, cols), lambda i: (i, 0))


def _pair_spec(cols):
    return pl.BlockSpec((ROW_BLK, cols), lambda i: (i, 0))


def _full_spec(rows, cols):
    return pl.BlockSpec((rows, cols), lambda i: (0, 0))


def _tc_call(body, in_specs, out_shape, out_specs):
    return pl.pallas_call(
        body,
        grid=(GRID,),
        in_specs=in_specs,
        out_shape=out_shape,
        out_specs=out_specs,
    )


def _enc_body(x_ref, w_ref, b_ref, y_ref):
    y = jnp.dot(x_ref[...], w_ref[...], preferred_element_type=jnp.float32)
    y_ref[...] = jnp.maximum(y + b_ref[...], 0.0)


def _stage1_body(xr, yr, w1r, dvr, ur):
    acc = jnp.dot(xr[...], w1r[:DH], preferred_element_type=jnp.float32)
    acc += jnp.dot(yr[...], w1r[DH:], preferred_element_type=jnp.float32)
    ur[...] = dvr[...] * acc


def _stage2_body(pa, pb, ur, dvr, b1r, w2pr, o1r, u1r):
    o1 = jnp.tanh(dvr[...] * (pa[...] + pb[...] + ur[...]) + b1r[...])
    o1r[...] = o1
    u1r[...] = dvr[...] * jnp.dot(o1, w2pr[...], preferred_element_type=jnp.float32)


def _stage3_body(pa, pb, u1r, dvr, b2r, cr, w3pr, v2r):
    o2 = jnp.tanh(dvr[...] * (pa[...] + pb[...] + u1r[...]) + b2r[...])
    v2r[...] = dvr[...] * (1.0 - o2 * o2) * (cr[...] * w3pr[...])


def _stage4_body(qa, qb, v2r, dvr, o1r, w2pr, v1r):
    t = dvr[...] * (qa[...] + qb[...] + v2r[...])
    go1 = jnp.dot(t, w2pr[...].T, preferred_element_type=jnp.float32)
    o1 = o1r[...]
    v1r[...] = dvr[...] * (1.0 - o1 * o1) * go1


def _stage5_body(qa, qb, v1r, dvr, w1r, xr, yr, xnr, ynr):
    z = dvr[...] * (qa[...] + qb[...] + v1r[...])
    xnr[...] = xr[...] + jnp.dot(z, w1r[DH:].T, preferred_element_type=jnp.float32)
    ynr[...] = yr[...] - jnp.dot(z, w1r[:DH].T, preferred_element_type=jnp.float32)


def _dec_body(xr, wr, br, outr):
    outr[...] = jnp.dot(xr[...], wr[...], preferred_element_type=jnp.float32) + br[...]


def kernel(x, edge_index, W_enc, b_enc, W1, b1, W2, b2, W3, b3, W_dec, b_dec):
    f32 = jnp.float32
    src = edge_index[0]
    dst = edge_index[1]
    z128 = jnp.zeros((RPT, 128), f32)
    ones128 = jnp.ones((N, 128), f32)

    npad = EPAD - E
    ppt = npad // NW
    ept = E // NW
    gpad = (jnp.arange(npad, dtype=jnp.int32) % N).reshape(NW, ppt)
    spad = (N + jnp.arange(npad, dtype=jnp.int32) % (NP - N)).reshape(NW, ppt)

    def _tile_layout(real, pad):
        return jnp.concatenate([real.reshape(NW, ept), pad], axis=1).reshape(-1)

    g_fwd = _tile_layout(src, gpad).reshape(TOTCH, KCH)
    s_fwd = _tile_layout(dst, spad).reshape(TOTCH, KCH)
    g_bwd = _tile_layout(dst, gpad).reshape(TOTCH, KCH)
    s_bwd = _tile_layout(src, spad).reshape(TOTCH, KCH)

    degp = _count128(ones128, g_fwd, s_fwd, z128)
    deg = degp[:N, 0] + degp[NP:NP + N, 0] + 1.0
    dinv = lax.rsqrt(deg)
    dinv128 = jnp.broadcast_to(dinv[:, None], (N, 128))
    ctp = _prop128(dinv128, g_bwd, s_bwd, z128)
    c0 = dinv * (ctp[:N, 0] + ctp[NP:NP + N, 0] + dinv)

    dv2 = dinv[:, None]
    c2 = c0[:, None]
    b1_ = b1[None, :]
    b2p = jnp.concatenate([b2, jnp.zeros((DH,), f32)])[None, :]
    benc_ = b_enc[None, :]
    bdec_ = b_dec[None, :]
    W2p = jnp.concatenate([W2, jnp.zeros((128, DH), f32)], axis=1)
    w3p = jnp.concatenate([W3[:, 0], jnp.zeros((DH,), f32)])[None, :]

    sc_dv = pl.BlockSpec((ROW_BLK, 1), lambda i: (i, 0))

    Y = _tc_call(
        _enc_body,
        [_row_spec(128), _full_spec(128, DH), _full_spec(1, DH)],
        jax.ShapeDtypeStruct((N, DH), f32),
        _row_spec(DH),
    )(x, W_enc, benc_)
    X = Y

    stage1 = _tc_call(
        _stage1_body,
        [_row_spec(DH), _row_spec(DH), _full_spec(128, 128), sc_dv],
        jax.ShapeDtypeStruct((N, 128), f32),
        _row_spec(128),
    )
    stage2 = _tc_call(
        _stage2_body,
        [_row_spec(128), _row_spec(128), _row_spec(128), sc_dv,
         _full_spec(1, 128), _full_spec(128, 128)],
        [jax.ShapeDtypeStruct((N, 128), f32), jax.ShapeDtypeStruct((N, 128), f32)],
        [_row_spec(128), _row_spec(128)],
    )
    stage3 = _tc_call(
        _stage3_body,
        [_row_spec(128), _row_spec(128), _row_spec(128), sc_dv,
         _full_spec(1, 128), sc_dv, _full_spec(1, 128)],
        jax.ShapeDtypeStruct((N, 128), f32),
        _row_spec(128),
    )
    stage4 = _tc_call(
        _stage4_body,
        [_row_spec(128), _row_spec(128), _row_spec(128), sc_dv,
         _row_spec(128), _full_spec(128, 128)],
        jax.ShapeDtypeStruct((N, 128), f32),
        _row_spec(128),
    )
    stage5 = _tc_call(
        _stage5_body,
        [_row_spec(128), _row_spec(128), _row_spec(128), sc_dv,
         _full_spec(128, 128), _row_spec(DH), _row_spec(DH)],
        [jax.ShapeDtypeStruct((N, DH), f32), jax.ShapeDtypeStruct((N, DH), f32)],
        [_row_spec(DH), _row_spec(DH)],
    )

    for _ in range(2):
        u0 = stage1(X, Y, W1, dv2)
        p0 = _prop128(u0, g_fwd, s_fwd, z128)
        o1, u1 = stage2(p0[:N], p0[NP:NP + N], u0, dv2, b1_, W2p)
        p1 = _prop128(u1, g_fwd, s_fwd, z128)
        v2 = stage3(p1[:N], p1[NP:NP + N], u1, dv2, b2p, c2, w3p)
        q2 = _prop128(v2, g_bwd, s_bwd, z128)
        v1 = stage4(q2[:N], q2[NP:NP + N], v2, dv2, o1, W2p)
        q1 = _prop128(v1, g_bwd, s_bwd, z128)
        X, Y = stage5(q1[:N], q1[NP:NP + N], v1, dv2, W1, X, Y)

    out = _tc_call(
        _dec_body,
        [_row_spec(DH), _full_spec(DH, 16), _full_spec(1, 16)],
        jax.ShapeDtypeStruct((N, 16), f32),
        _row_spec(16),
    )(X, W_dec, bdec_)
    return out


# speedup vs baseline: 4.8410x; 1.0134x over previous
import functools

import jax
import jax.numpy as jnp
from jax import lax
from jax.experimental import pallas as pl
from jax.experimental.pallas import tpu as pltpu
from jax.experimental.pallas import tpu_sc as plsc

N = 10000
E = 320000
DH = 64
NC = 2
NS = 16
NW = NC * NS
KCH = 128
NCHUNK = 80
PH = 2
CPP = NCHUNK // PH
NH = CPP // 2
EPAD = NW * NCHUNK * KCH
TOTCH = EPAD // KCH
NP = N + 496
RPT = NP // NS

ROW_BLK = 2000
GRID = N // ROW_BLK


def _make_prop(d, gather=True):
    mesh = plsc.VectorSubcoreMesh(
        core_axis_name="c", subcore_axis_name="s", num_cores=NC, num_subcores=NS
    )

    @functools.partial(
        pl.kernel,
        out_type=jax.ShapeDtypeStruct((NC * NP, d), jnp.float32),
        mesh=mesh,
        scratch_types=[
            pltpu.VMEM_SHARED((NP, d), jnp.float32),
            pltpu.VMEM((CPP, KCH), jnp.int32),
            pltpu.VMEM((CPP, KCH), jnp.int32),
            pltpu.VMEM((KCH, d), jnp.float32),
            pltpu.VMEM((KCH, d), jnp.float32),
            pltpu.SemaphoreType.DMA,
            pltpu.SemaphoreType.DMA,
            pltpu.SemaphoreType.DMA,
            pltpu.SemaphoreType.DMA,
        ],
    )
    def prop(table, idxg, idxs, zeros, out, acc, gidx_v, sidx_v, rows_a, rows_b,
             sa, sb, pa, pb):
        cid = lax.axis_index("c")
        sid = lax.axis_index("s")
        wid = cid * NS + sid
        r0 = sid * RPT
        pltpu.sync_copy(zeros, acc.at[pl.ds(r0, RPT)])
        plsc.subcore_barrier()

        c0 = wid * NCHUNK

        def _fire_g(i, rows, sem):
            pltpu.async_copy(table.at[gidx_v.at[i]], rows, sem)

        def _wait_g(i, rows, sem):
            pltpu.make_async_copy(table.at[gidx_v.at[i]], rows, sem).wait()

        def _fire_s(i, rows, sem):
            pltpu.async_copy(rows, acc.at[sidx_v.at[i]], sem, add=True)

        def _wait_s(i, rows, sem):
            pltpu.make_async_copy(rows, acc.at[sidx_v.at[i]], sem).wait()

        if not gather:
            pltpu.sync_copy(table.at[pl.ds(0, KCH)], rows_a)
            pltpu.sync_copy(table.at[pl.ds(0, KCH)], rows_b)

            def cbody(j, carry):
                pltpu.sync_copy(idxs.at[pl.ds(c0 + j, 1)], sidx_v.at[pl.ds(0, 1)])
                pltpu.sync_copy(rows_a, acc.at[sidx_v.at[0]], add=True)
                return carry

            for p in range(PH):
                pltpu.sync_copy(idxs.at[pl.ds(c0 + p * CPP, CPP)], sidx_v)

                def cbody2(j, carry):
                    pltpu.sync_copy(rows_a, acc.at[sidx_v.at[j]], add=True)
                    return carry

                lax.fori_loop(0, CPP, cbody2, 0)
            plsc.subcore_barrier()
            pltpu.sync_copy(
                acc.at[pl.ds(r0, RPT)], out.at[pl.ds(cid * NP + r0, RPT)]
            )
            return

        for p in range(PH):
            pltpu.sync_copy(idxg.at[pl.ds(c0 + p * CPP, CPP)], gidx_v)
            pltpu.sync_copy(idxs.at[pl.ds(c0 + p * CPP, CPP)], sidx_v)
            _fire_g(0, rows_a, sa)
            _wait_g(0, rows_a, sa)
            _fire_s(0, rows_a, pa)
            _fire_g(1, rows_b, sb)
            _wait_g(1, rows_b, sb)
            _fire_s(1, rows_b, pb)
            _wait_s(0, rows_a, pa)
            _fire_g(2, rows_a, sa)

            def body(j, carry):
                i0 = 2 * j
                i1 = i0 + 1
                _wait_g(i0, rows_a, sa)
                _fire_s(i0, rows_a, pa)
                _wait_s(i1 - 2, rows_b, pb)
                _fire_g(i1, rows_b, sb)
                _wait_g(i1, rows_b, sb)
                _fire_s(i1, rows_b, pb)
                _wait_s(i0, rows_a, pa)
                _fire_g(i0 + 2, rows_a, sa)
                return carry

            lax.fori_loop(1, NH - 1, body, 0)

            i0 = CPP - 2
            i1 = CPP - 1
            _wait_g(i0, rows_a, sa)
            _fire_s(i0, rows_a, pa)
            _wait_s(i1 - 2, rows_b, pb)
            _fire_g(i1, rows_b, sb)
            _wait_g(i1, rows_b, sb)
            _fire_s(i1, rows_b, pb)
            _wait_s(i0, rows_a, pa)
            _wait_s(i1, rows_b, pb)

        plsc.subcore_barrier()
        pltpu.sync_copy(
            acc.at[pl.ds(r0, RPT)], out.at[pl.ds(cid * NP + r0, RPT)]
        )

    return prop


_prop128 = _make_prop(128)
_count128 = _make_prop(128, gather=False)


def _row_spec(cols):
    return pl.BlockSpec((ROW_BLK, cols), lambda i: (i, 0))


def _pair_spec(cols):
    return pl.BlockSpec((ROW_BLK, cols), lambda i: (i, 0))


def _full_spec(rows, cols):
    return pl.BlockSpec((rows, cols), lambda i: (0, 0))


def _tc_call(body, in_specs, out_shape, out_specs):
    return pl.pallas_call(
        body,
        grid=(GRID,),
        in_specs=in_specs,
        out_shape=out_shape,
        out_specs=out_specs,
    )


def _enc1_body(x_ref, w_ref, b_ref, w1s_ref, dvr, y_ref, ur):
    y = jnp.dot(x_ref[...], w_ref[...], preferred_element_type=jnp.float32)
    y = jnp.maximum(y + b_ref[...], 0.0)
    y_ref[...] = y
    ur[...] = dvr[...] * jnp.dot(y, w1s_ref[...], preferred_element_type=jnp.float32)


def _stage1_body(xr, yr, w1r, dvr, ur):
    acc = jnp.dot(xr[...], w1r[:DH], preferred_element_type=jnp.float32)
    acc += jnp.dot(yr[...], w1r[DH:], preferred_element_type=jnp.float32)
    ur[...] = dvr[...] * acc


def _stage2_body(pa, pb, ur, dvr, b1r, w2pr, o1r, u1r):
    o1 = jnp.tanh(dvr[...] * (pa[...] + pb[...] + ur[...]) + b1r[...])
    o1r[...] = o1
    u1r[...] = dvr[...] * jnp.dot(o1, w2pr[...], preferred_element_type=jnp.float32)


def _stage3_body(pa, pb, u1r, dvr, b2r, cr, w3pr, v2r):
    o2 = jnp.tanh(dvr[...] * (pa[...] + pb[...] + u1r[...]) + b2r[...])
    v2r[...] = dvr[...] * (1.0 - o2 * o2) * (cr[...] * w3pr[...])


def _stage4_body(qa, qb, v2r, dvr, o1r, w2pr, v1r):
    t = dvr[...] * (qa[...] + qb[...] + v2r[...])
    go1 = jnp.dot(t, w2pr[...].T, preferred_element_type=jnp.float32)
    o1 = o1r[...]
    v1r[...] = dvr[...] * (1.0 - o1 * o1) * go1


def _stage51_body(qa, qb, v1r, dvr, w1r, xr, yr, xnr, ynr, ur):
    z = dvr[...] * (qa[...] + qb[...] + v1r[...])
    xn = xr[...] + jnp.dot(z, w1r[DH:].T, preferred_element_type=jnp.float32)
    yn = yr[...] - jnp.dot(z, w1r[:DH].T, preferred_element_type=jnp.float32)
    xnr[...] = xn
    ynr[...] = yn
    acc = jnp.dot(xn, w1r[:DH], preferred_element_type=jnp.float32)
    acc += jnp.dot(yn, w1r[DH:], preferred_element_type=jnp.float32)
    ur[...] = dvr[...] * acc


def _stage5dec_body(qa, qb, v1r, dvr, w1r, xr, wdr, bdr, outr):
    z = dvr[...] * (qa[...] + qb[...] + v1r[...])
    xn = xr[...] + jnp.dot(z, w1r[DH:].T, preferred_element_type=jnp.float32)
    outr[...] = jnp.dot(xn, wdr[...], preferred_element_type=jnp.float32) + bdr[...]


def _dec_body(xr, wr, br, outr):
    outr[...] = jnp.dot(xr[...], wr[...], preferred_element_type=jnp.float32) + br[...]


def kernel(x, edge_index, W_enc, b_enc, W1, b1, W2, b2, W3, b3, W_dec, b_dec):
    f32 = jnp.float32
    src = edge_index[0]
    dst = edge_index[1]
    z128 = jnp.zeros((RPT, 128), f32)
    ones128 = jnp.ones((N, 128), f32)

    npad = EPAD - E
    ppt = npad // NW
    ept = E // NW
    gpad = (jnp.arange(npad, dtype=jnp.int32) % N).reshape(NW, ppt)
    spad = (N + jnp.arange(npad, dtype=jnp.int32) % (NP - N)).reshape(NW, ppt)

    def _tile_layout(real, pad):
        return jnp.concatenate([real.reshape(NW, ept), pad], axis=1).reshape(-1)

    g_fwd = _tile_layout(src, gpad).reshape(TOTCH, KCH)
    s_fwd = _tile_layout(dst, spad).reshape(TOTCH, KCH)
    g_bwd = _tile_layout(dst, gpad).reshape(TOTCH, KCH)
    s_bwd = _tile_layout(src, spad).reshape(TOTCH, KCH)

    degp = _count128(ones128, g_fwd, s_fwd, z128)
    deg = degp[:N, 0] + degp[NP:NP + N, 0] + 1.0
    dinv = lax.rsqrt(deg)
    dinv128 = jnp.broadcast_to(dinv[:, None], (N, 128))
    ctp = _prop128(dinv128, g_bwd, s_bwd, z128)
    c0 = dinv * (ctp[:N, 0] + ctp[NP:NP + N, 0] + dinv)

    dv2 = dinv[:, None]
    c2 = c0[:, None]
    b1_ = b1[None, :]
    b2p = jnp.concatenate([b2, jnp.zeros((DH,), f32)])[None, :]
    benc_ = b_enc[None, :]
    bdec_ = b_dec[None, :]
    W2p = jnp.concatenate([W2, jnp.zeros((128, DH), f32)], axis=1)
    w3p = jnp.concatenate([W3[:, 0], jnp.zeros((DH,), f32)])[None, :]

    sc_dv = pl.BlockSpec((ROW_BLK, 1), lambda i: (i, 0))

    enc1 = _tc_call(
        _enc1_body,
        [_row_spec(128), _full_spec(128, DH), _full_spec(1, DH),
         _full_spec(DH, 128), sc_dv],
        [jax.ShapeDtypeStruct((N, DH), f32), jax.ShapeDtypeStruct((N, 128), f32)],
        [_row_spec(DH), _row_spec(128)],
    )
    stage2 = _tc_call(
        _stage2_body,
        [_row_spec(128), _row_spec(128), _row_spec(128), sc_dv,
         _full_spec(1, 128), _full_spec(128, 128)],
        [jax.ShapeDtypeStruct((N, 128), f32), jax.ShapeDtypeStruct((N, 128), f32)],
        [_row_spec(128), _row_spec(128)],
    )
    stage3 = _tc_call(
        _stage3_body,
        [_row_spec(128), _row_spec(128), _row_spec(128), sc_dv,
         _full_spec(1, 128), sc_dv, _full_spec(1, 128)],
        jax.ShapeDtypeStruct((N, 128), f32),
        _row_spec(128),
    )
    stage4 = _tc_call(
        _stage4_body,
        [_row_spec(128), _row_spec(128), _row_spec(128), sc_dv,
         _row_spec(128), _full_spec(128, 128)],
        jax.ShapeDtypeStruct((N, 128), f32),
        _row_spec(128),
    )
    stage51 = _tc_call(
        _stage51_body,
        [_row_spec(128), _row_spec(128), _row_spec(128), sc_dv,
         _full_spec(128, 128), _row_spec(DH), _row_spec(DH)],
        [jax.ShapeDtypeStruct((N, DH), f32), jax.ShapeDtypeStruct((N, DH), f32),
         jax.ShapeDtypeStruct((N, 128), f32)],
        [_row_spec(DH), _row_spec(DH), _row_spec(128)],
    )
    stage5dec = _tc_call(
        _stage5dec_body,
        [_row_spec(128), _row_spec(128), _row_spec(128), sc_dv,
         _full_spec(128, 128), _row_spec(DH), _full_spec(DH, 16), _full_spec(1, 16)],
        jax.ShapeDtypeStruct((N, 16), f32),
        _row_spec(16),
    )

    W1sum = W1[:DH] + W1[DH:]
    Y, u0 = enc1(x, W_enc, benc_, W1sum, dv2)
    X = Y
    for it in range(2):
        p0 = _prop128(u0, g_fwd, s_fwd, z128)
        o1, u1 = stage2(p0[:N], p0[NP:NP + N], u0, dv2, b1_, W2p)
        p1 = _prop128(u1, g_fwd, s_fwd, z128)
        v2 = stage3(p1[:N], p1[NP:NP + N], u1, dv2, b2p, c2, w3p)
        q2 = _prop128(v2, g_bwd, s_bwd, z128)
        v1 = stage4(q2[:N], q2[NP:NP + N], v2, dv2, o1, W2p)
        q1 = _prop128(v1, g_bwd, s_bwd, z128)
        if it == 0:
            X, Y, u0 = stage51(q1[:N], q1[NP:NP + N], v1, dv2, W1, X, Y)
        else:
            out = stage5dec(q1[:N], q1[NP:NP + N], v1, dv2, W1, X, W_dec, bdec_)
    return out


# speedup vs baseline: 4.8511x; 1.0021x over previous
import functools

import jax
import jax.numpy as jnp
from jax import lax
from jax.experimental import pallas as pl
from jax.experimental.pallas import tpu as pltpu
from jax.experimental.pallas import tpu_sc as plsc

N = 10000
E = 320000
DH = 64
NC = 2
NS = 16
NW = NC * NS
KCH = 128
NCHUNK = 80
PH = 2
CPP = NCHUNK // PH
NH = CPP // 2
EPAD = NW * NCHUNK * KCH
TOTCH = EPAD // KCH
NP = N + 496
RPT = NP // NS

ROW_BLK = 2000
GRID = N // ROW_BLK


def _make_prop(d, gather=True):
    mesh = plsc.VectorSubcoreMesh(
        core_axis_name="c", subcore_axis_name="s", num_cores=NC, num_subcores=NS
    )

    @functools.partial(
        pl.kernel,
        out_type=jax.ShapeDtypeStruct((NC * NP, d), jnp.float32),
        mesh=mesh,
        scratch_types=[
            pltpu.VMEM_SHARED((NP, d), jnp.float32),
            pltpu.VMEM((CPP, KCH), jnp.int32),
            pltpu.VMEM((CPP, KCH), jnp.int32),
            pltpu.VMEM((KCH, d), jnp.float32),
            pltpu.VMEM((KCH, d), jnp.float32),
            pltpu.SemaphoreType.DMA,
            pltpu.SemaphoreType.DMA,
            pltpu.SemaphoreType.DMA,
            pltpu.SemaphoreType.DMA,
        ],
    )
    def prop(table, idxg, idxs, zeros, out, acc, gidx_v, sidx_v, rows_a, rows_b,
             sa, sb, pa, pb):
        cid = lax.axis_index("c")
        sid = lax.axis_index("s")
        wid = cid * NS + sid
        r0 = sid * RPT
        pltpu.sync_copy(zeros, acc.at[pl.ds(r0, RPT)])
        plsc.subcore_barrier()

        c0 = wid * NCHUNK

        def _fire_g(i, rows, sem):
            pltpu.async_copy(table.at[gidx_v.at[i]], rows, sem)

        def _wait_g(i, rows, sem):
            pltpu.make_async_copy(table.at[gidx_v.at[i]], rows, sem).wait()

        def _fire_s(i, rows, sem):
            pltpu.async_copy(rows, acc.at[sidx_v.at[i]], sem, add=True)

        def _wait_s(i, rows, sem):
            pltpu.make_async_copy(rows, acc.at[sidx_v.at[i]], sem).wait()

        if not gather:
            pltpu.sync_copy(table.at[pl.ds(0, KCH)], rows_a)
            pltpu.sync_copy(table.at[pl.ds(0, KCH)], rows_b)

            for p in range(PH):
                pltpu.sync_copy(idxs.at[pl.ds(c0 + p * CPP, CPP)], sidx_v)

                def cbody2(j, carry):
                    pltpu.sync_copy(rows_a, acc.at[sidx_v.at[j]], add=True)
                    return carry

                lax.fori_loop(0, CPP, cbody2, 0)
            plsc.subcore_barrier()
            pltpu.sync_copy(
                acc.at[pl.ds(r0, RPT)], out.at[pl.ds(cid * NP + r0, RPT)]
            )
            return

        for p in range(PH):
            pltpu.sync_copy(idxg.at[pl.ds(c0 + p * CPP, CPP)], gidx_v)
            pltpu.sync_copy(idxs.at[pl.ds(c0 + p * CPP, CPP)], sidx_v)
            _fire_g(0, rows_a, sa)
            _wait_g(0, rows_a, sa)
            _fire_s(0, rows_a, pa)
            _fire_g(1, rows_b, sb)
            _wait_g(1, rows_b, sb)
            _fire_s(1, rows_b, pb)
            _wait_s(0, rows_a, pa)
            _fire_g(2, rows_a, sa)

            def body(j, carry):
                i0 = 2 * j
                i1 = i0 + 1
                _wait_g(i0, rows_a, sa)
                _fire_s(i0, rows_a, pa)
                _wait_s(i1 - 2, rows_b, pb)
                _fire_g(i1, rows_b, sb)
                _wait_g(i1, rows_b, sb)
                _fire_s(i1, rows_b, pb)
                _wait_s(i0, rows_a, pa)
                _fire_g(i0 + 2, rows_a, sa)
                return carry

            lax.fori_loop(1, NH - 1, body, 0)

            i0 = CPP - 2
            i1 = CPP - 1
            _wait_g(i0, rows_a, sa)
            _fire_s(i0, rows_a, pa)
            _wait_s(i1 - 2, rows_b, pb)
            _fire_g(i1, rows_b, sb)
            _wait_g(i1, rows_b, sb)
            _fire_s(i1, rows_b, pb)
            _wait_s(i0, rows_a, pa)
            _wait_s(i1, rows_b, pb)

        plsc.subcore_barrier()
        pltpu.sync_copy(
            acc.at[pl.ds(r0, RPT)], out.at[pl.ds(cid * NP + r0, RPT)]
        )

    return prop


_prop128 = _make_prop(128)
_count128 = _make_prop(128, gather=False)


def _row_spec(cols):
    return pl.BlockSpec((ROW_BLK, cols), lambda i: (i, 0))


def _full_spec(rows, cols):
    return pl.BlockSpec((rows, cols), lambda i: (0, 0))


def _tc_call(body, in_specs, out_shape, out_specs):
    return pl.pallas_call(
        body,
        grid=(GRID,),
        in_specs=in_specs,
        out_shape=out_shape,
        out_specs=out_specs,
    )


def _enc1_body(x_ref, w_ref, b_ref, w1s_ref, dvr, y_ref, ur):
    y = jnp.dot(x_ref[...], w_ref[...], preferred_element_type=jnp.float32)
    y = jnp.maximum(y + b_ref[...], 0.0)
    y_ref[...] = y
    ur[...] = dvr[...] * jnp.dot(y, w1s_ref[...], preferred_element_type=jnp.float32)


def _stage2_body(pa, pb, ur, dvr, b1r, w2pr, o1r, u1r):
    o1 = jnp.tanh(dvr[...] * (pa[...] + pb[...] + ur[...]) + b1r[...])
    o1r[...] = o1
    u1r[...] = dvr[...] * jnp.dot(o1, w2pr[...], preferred_element_type=jnp.float32)


def _stage3_body(pa, pb, u1r, dvr, b2r, cr, w3pr, v2r):
    o2 = jnp.tanh(dvr[...] * (pa[...] + pb[...] + u1r[...]) + b2r[...])
    v2r[...] = dvr[...] * (1.0 - o2 * o2) * (cr[...] * w3pr[...])


def _stage4_body(qa, qb, v2r, dvr, o1r, w2pr, v1r):
    t = dvr[...] * (qa[...] + qb[...] + v2r[...])
    go1 = jnp.dot(t, w2pr[...].T, preferred_element_type=jnp.float32)
    o1 = o1r[...]
    v1r[...] = dvr[...] * (1.0 - o1 * o1) * go1


def _stage51_body(qa, qb, v1r, dvr, w1r, xr, yr, xnr, ynr, ur):
    z = dvr[...] * (qa[...] + qb[...] + v1r[...])
    xn = xr[...] + jnp.dot(z, w1r[DH:].T, preferred_element_type=jnp.float32)
    yn = yr[...] - jnp.dot(z, w1r[:DH].T, preferred_element_type=jnp.float32)
    xnr[...] = xn
    ynr[...] = yn
    acc = jnp.dot(xn, w1r[:DH], preferred_element_type=jnp.float32)
    acc += jnp.dot(yn, w1r[DH:], preferred_element_type=jnp.float32)
    ur[...] = dvr[...] * acc


def _stage5dec_body(qa, qb, v1r, dvr, w1r, xr, wdr, bdr, outr):
    z = dvr[...] * (qa[...] + qb[...] + v1r[...])
    xn = xr[...] + jnp.dot(z, w1r[DH:].T, preferred_element_type=jnp.float32)
    outr[...] = jnp.dot(xn, wdr[...], preferred_element_type=jnp.float32) + bdr[...]


def kernel(x, edge_index, W_enc, b_enc, W1, b1, W2, b2, W3, b3, W_dec, b_dec):
    f32 = jnp.float32
    src = edge_index[0]
    dst = edge_index[1]
    z128 = jnp.zeros((RPT, 128), f32)
    ones128 = jnp.ones((N, 128), f32)

    npad = EPAD - E
    ppt = npad // NW
    ept = E // NW
    gpad = (jnp.arange(npad, dtype=jnp.int32) % N).reshape(NW, ppt)
    spad = (N + jnp.arange(npad, dtype=jnp.int32) % (NP - N)).reshape(NW, ppt)

    def _tile_layout(real, pad):
        return jnp.concatenate([real.reshape(NW, ept), pad], axis=1).reshape(-1)

    g_fwd = _tile_layout(src, gpad).reshape(TOTCH, KCH)
    s_fwd = _tile_layout(dst, spad).reshape(TOTCH, KCH)
    g_bwd = _tile_layout(dst, gpad).reshape(TOTCH, KCH)
    s_bwd = _tile_layout(src, spad).reshape(TOTCH, KCH)

    degp = _count128(ones128, g_fwd, s_fwd, z128)
    deg = degp[:N, 0] + degp[NP:NP + N, 0] + 1.0
    dinv = lax.rsqrt(deg)
    dinv128 = jnp.broadcast_to(dinv[:, None], (N, 128))
    ctp = _prop128(dinv128, g_bwd, s_bwd, z128)
    c0 = dinv * (ctp[:N, 0] + ctp[NP:NP + N, 0] + dinv)

    dv2 = dinv[:, None]
    c2 = c0[:, None]
    b1_ = b1[None, :]
    b2p = jnp.concatenate([b2, jnp.zeros((DH,), f32)])[None, :]
    benc_ = b_enc[None, :]
    bdec_ = b_dec[None, :]
    W2p = jnp.concatenate([W2, jnp.zeros((128, DH), f32)], axis=1)
    w3p = jnp.concatenate([W3[:, 0], jnp.zeros((DH,), f32)])[None, :]

    sc_dv = pl.BlockSpec((ROW_BLK, 1), lambda i: (i, 0))

    enc1 = _tc_call(
        _enc1_body,
        [_row_spec(128), _full_spec(128, DH), _full_spec(1, DH),
         _full_spec(DH, 128), sc_dv],
        [jax.ShapeDtypeStruct((N, DH), f32), jax.ShapeDtypeStruct((N, 128), f32)],
        [_row_spec(DH), _row_spec(128)],
    )
    stage2 = _tc_call(
        _stage2_body,
        [_row_spec(128), _row_spec(128), _row_spec(128), sc_dv,
         _full_spec(1, 128), _full_spec(128, 128)],
        [jax.ShapeDtypeStruct((N, 128), f32), jax.ShapeDtypeStruct((N, 128), f32)],
        [_row_spec(128), _row_spec(128)],
    )
    stage3 = _tc_call(
        _stage3_body,
        [_row_spec(128), _row_spec(128), _row_spec(128), sc_dv,
         _full_spec(1, 128), sc_dv, _full_spec(1, 128)],
        jax.ShapeDtypeStruct((N, 128), f32),
        _row_spec(128),
    )
    stage4 = _tc_call(
        _stage4_body,
        [_row_spec(128), _row_spec(128), _row_spec(128), sc_dv,
         _row_spec(128), _full_spec(128, 128)],
        jax.ShapeDtypeStruct((N, 128), f32),
        _row_spec(128),
    )
    stage51 = _tc_call(
        _stage51_body,
        [_row_spec(128), _row_spec(128), _row_spec(128), sc_dv,
         _full_spec(128, 128), _row_spec(DH), _row_spec(DH)],
        [jax.ShapeDtypeStruct((N, DH), f32), jax.ShapeDtypeStruct((N, DH), f32),
         jax.ShapeDtypeStruct((N, 128), f32)],
        [_row_spec(DH), _row_spec(DH), _row_spec(128)],
    )
    stage5dec = _tc_call(
        _stage5dec_body,
        [_row_spec(128), _row_spec(128), _row_spec(128), sc_dv,
         _full_spec(128, 128), _row_spec(DH), _full_spec(DH, 16), _full_spec(1, 16)],
        jax.ShapeDtypeStruct((N, 16), f32),
        _row_spec(16),
    )

    W1sum = W1[:DH] + W1[DH:]
    Y, u0 = enc1(x, W_enc, benc_, W1sum, dv2)
    X = Y
    for it in range(2):
        p0 = _prop128(u0, g_fwd, s_fwd, z128)
        o1, u1 = stage2(p0[:N], p0[NP:NP + N], u0, dv2, b1_, W2p)
        p1 = _prop128(u1, g_fwd, s_fwd, z128)
        v2 = stage3(p1[:N], p1[NP:NP + N], u1, dv2, b2p, c2, w3p)
        q2 = _prop128(v2, g_bwd, s_bwd, z128)
        v1 = stage4(q2[:N], q2[NP:NP + N], v2, dv2, o1, W2p)
        q1 = _prop128(v1, g_bwd, s_bwd, z128)
        if it == 0:
            X, Y, u0 = stage51(q1[:N], q1[NP:NP + N], v1, dv2, W1, X, Y)
        else:
            out = stage5dec(q1[:N], q1[NP:NP + N], v1, dv2, W1, X, W_dec, bdec_)
    return out
